# baseline (speedup 1.0000x reference)
import functools

import jax
import jax.numpy as jnp
from jax import lax
from jax.experimental import pallas as pl
from jax.experimental.pallas import tpu as pltpu
from jax.experimental.pallas import tpu_sc as plsc

_N = 10000
_D = 128
_E = 160000
_R = 10
_CH = 128
_NCH = _E // _CH
_SIDE = _R * _E

_RELS = ['drug_drug', 'drug_disease', 'disease_disease', 'drug_protein',
         'protein_protein', 'protein_gene', 'gene_gene', 'gene_pathway',
         'pathway_pathway', 'pathway_disease']
_SRC_T = [0, 0, 1, 0, 2, 2, 3, 3, 4, 4]

_ZBLOCK = ['dd', 'dp', 'dd', 'pd', 'dp', 'pg', 'pg', 'gp', 'gp', 'pd']
_CONTRIB = [
    [(0, 'drug_drug')],
    [(0, 'drug_drug')],
    [(1, 'drug_disease'), (2, 'disease_disease')],
    [(9, 'pathway_disease'), (2, 'disease_disease')],
    [(3, 'drug_protein'), (4, 'protein_protein')],
    [(4, 'protein_protein')],
    [(5, 'protein_gene'), (6, 'gene_gene')],
    [(6, 'gene_gene')],
    [(7, 'gene_pathway'), (8, 'pathway_pathway')],
    [(8, 'pathway_pathway')],
]
_WLIST = []
_CONTRIB_IDX = []
for _zi, _lst in enumerate(_CONTRIB):
    _idxs = []
    for (_r, _rl) in _lst:
        _idxs.append((_r, len(_WLIST)))
        _WLIST.append((_ZBLOCK[_zi], _rl))
    _CONTRIB_IDX.append(_idxs)


_CE = 2000


def _deg_body(eraw, degp, bins, idxb0, idxb1, sem_i0, sem_i1):
    idxb = [idxb0, idxb1]
    c = lax.axis_index("c")
    s = lax.axis_index("s")
    zero16 = jnp.zeros((16,), jnp.float32)
    ones16 = jnp.ones((16,), jnp.float32)
    sems = [sem_i0, sem_i1]

    @pl.loop(0, _R * _N // 16)
    def _zero(j):
        bins[pl.ds(j * 16, 16)] = zero16

    nch = _R * (_E // _CE) // 16

    def _issue(t, b):
        r = t // 5
        ch = s + 16 * (t % 5)
        off = (2 * r + c) * _E + ch * _CE
        pltpu.async_copy(eraw.at[pl.ds(off, _CE)], idxb[b], sems[b])

    _issue(0, 0)
    _issue(1, 1)

    @pl.loop(0, nch // 2)
    def _chunks(g):
        for b in range(2):
            t = g * 2 + b
            rn = (t // 5) * _N
            pltpu.make_async_copy(eraw.at[pl.ds(0, _CE)], idxb[b],
                                  sems[b]).wait()

            @pl.loop(0, _CE // 16, unroll=4)
            def _scat(j):
                v = idxb[b][pl.ds(j * 16, 16)] + rn
                plsc.addupdate_scatter(bins, [v], ones16)

            @pl.when(t + 2 < nch)
            def _():
                _issue(t + 2, b)

    w = c * 16 + s
    for r in range(_R):
        pltpu.sync_copy(bins.at[pl.ds(r * _N, _N)],
                        degp.at[pl.ds((r * 32 + w) * _N, _N)])


def _run_deg(ecat):
    mesh = plsc.VectorSubcoreMesh(core_axis_name="c", subcore_axis_name="s",
                                  num_cores=2, num_subcores=16)
    return pl.kernel(
        _deg_body,
        out_type=jax.ShapeDtypeStruct((32 * _R * _N,), jnp.float32),
        mesh=mesh,
        compiler_params=pltpu.CompilerParams(needs_layout_passes=False),
        scratch_types=[
            pltpu.VMEM((_R * _N,), jnp.float32),
            pltpu.VMEM((_CE,), jnp.int32),
            pltpu.VMEM((_CE,), jnp.int32),
            pltpu.SemaphoreType.DMA,
            pltpu.SemaphoreType.DMA,
        ],
    )(ecat)


def _norm_body(x_ref, deg_ref, xs_ref, inn_ref):
    deg = deg_ref[...]
    outd = jnp.sum(deg[0, 0, :, :], axis=0)
    ind = jnp.sum(deg[0, 1, :, :], axis=0)
    onorm = lax.rsqrt(jnp.maximum(outd, 1.0))
    inorm = lax.rsqrt(jnp.maximum(ind, 1.0))
    xs_ref[0] = x_ref[0] * onorm[:, None]
    inn_ref[0, 0] = inorm


def _src_t_of(r):
    return jnp.where(r >= 4, r // 2, jnp.where(r == 2, 1, 0))


def _run_norm(x5, deg4):
    return pl.pallas_call(
        _norm_body,
        grid=(_R,),
        in_specs=[
            pl.BlockSpec((1, _N, _D), lambda r: (_src_t_of(r), 0, 0)),
            pl.BlockSpec((1, 2, 16, _N), lambda r: (r, 0, 0, 0)),
        ],
        out_specs=[
            pl.BlockSpec((1, _N, _D), lambda r: (r, 0, 0)),
            pl.BlockSpec((1, 1, _N), lambda r: (r, 0, 0)),
        ],
        out_shape=[
            jax.ShapeDtypeStruct((_R, _N, _D), jnp.float32),
            jax.ShapeDtypeStruct((_R, 1, _N), jnp.float32),
        ],
    )(x5, deg4)


def _agg_body(xsf, eraw, agg, acc, idx2, rows, zrow,
              sem_i0, sem_i1, sem_i2, sem_i3, sem_g0, sem_g1,
              sem_s0, sem_s1):
    c = lax.axis_index("c")
    s = lax.axis_index("s")
    zero16 = jnp.zeros((16,), jnp.float32)
    sems_i = [sem_i0, sem_i1, sem_i2, sem_i3]
    sems_g = [sem_g0, sem_g1]
    sems_s = [sem_s0, sem_s1]

    @pl.loop(0, 80)
    def _zz(j):
        for t in range(8):
            zrow[j, pl.ds(t * 16, 16)] = zero16

    def _zero_stripe():
        @pl.loop(0, 8)
        def _zs(t):
            g = s + 16 * t

            @pl.when(g < 125)
            def _():
                pltpu.sync_copy(zrow, acc.at[pl.ds(g * 80, 80)])

    _zero_stripe()
    for i in range(_R // 2):
        r = i * 2 + c
        plsc.subcore_barrier()

        rn = r * _N

        def _issue_idx(t, bi):
            @pl.when(s + 16 * t < _NCH)
            def _():
                off = 2 * r * _E + (s + 16 * t) * _CH
                pltpu.async_copy(eraw.at[pl.ds(off, _CH)], idx2.at[bi, 0],
                                 sems_i[bi])
                pltpu.async_copy(eraw.at[pl.ds(off + _E, _CH)],
                                 idx2.at[bi, 1], sems_i[bi])

        def _wait_adjust_idx(t, bi):
            @pl.when(s + 16 * t < _NCH)
            def _():
                pltpu.make_async_copy(eraw.at[pl.ds(0, _CH)],
                                      idx2.at[bi, 0], sems_i[bi]).wait()
                pltpu.make_async_copy(eraw.at[pl.ds(0, _CH)],
                                      idx2.at[bi, 1], sems_i[bi]).wait()
                for j in range(8):
                    sl = pl.ds(j * 16, 16)
                    idx2[bi, 0, sl] = idx2[bi, 0, sl] + rn

        def _issue_gather(t, bi, bg):
            @pl.when(s + 16 * t < _NCH)
            def _():
                pltpu.async_copy(xsf.at[idx2.at[bi, 0]], rows.at[bg],
                                 sems_g[bg])

        def _wait_gather(t, bg):
            @pl.when(s + 16 * t < _NCH)
            def _():
                pltpu.make_async_copy(xsf.at[pl.ds(0, _CH)], rows.at[bg],
                                      sems_g[bg]).wait()

        def _issue_scatter(t, bi, bg):
            @pl.when(s + 16 * t < _NCH)
            def _():
                pltpu.async_copy(rows.at[bg], acc.at[idx2.at[bi, 1]],
                                 sems_s[bg], add=True)

        def _wait_scatter(t, bg, extra_cond):
            @pl.when(jnp.logical_and(extra_cond, s + 16 * t < _NCH))
            def _():
                pltpu.make_async_copy(xsf.at[pl.ds(0, _CH)], rows.at[bg],
                                      sems_s[bg]).wait()

        _issue_idx(0, 0)
        _issue_idx(1, 1)
        _issue_idx(2, 2)
        _wait_adjust_idx(0, 0)
        _issue_gather(0, 0, 0)

        @pl.loop(0, 21)
        def _chunks(g):
            for u in range(4):
                t = g * 4 + u
                _wait_adjust_idx(t + 1, (u + 1) % 4)
                _wait_gather(t, u % 2)
                _wait_scatter(t - 1, (u + 1) % 2, t >= 1)
                _issue_gather(t + 1, (u + 1) % 4, (u + 1) % 2)
                _issue_scatter(t, u % 4, u % 2)
                _issue_idx(t + 3, (u + 3) % 4)

        plsc.subcore_barrier()

        @pl.loop(0, 8)
        def _drain(t):
            g = s + 16 * t

            @pl.when(g < 125)
            def _():
                sl = pl.ds(g * 80, 80)
                pltpu.sync_copy(acc.at[sl], agg.at[r, sl])

        if i < _R // 2 - 1:
            _zero_stripe()


def _run_agg(xsf, eraw):
    mesh = plsc.VectorSubcoreMesh(core_axis_name="c", subcore_axis_name="s",
                                  num_cores=2, num_subcores=16)
    return pl.kernel(
        _agg_body,
        out_type=jax.ShapeDtypeStruct((_R, _N, _D), jnp.float32),
        mesh=mesh,
        compiler_params=pltpu.CompilerParams(needs_layout_passes=False),
        scratch_types=[
            pltpu.VMEM_SHARED((_N, _D), jnp.float32),
            pltpu.VMEM((4, 2, _CH), jnp.int32),
            pltpu.VMEM((2, _CH, _D), jnp.float32),
            pltpu.VMEM((80, _D), jnp.float32),
        ] + [pltpu.SemaphoreType.DMA] * 8,
    )(xsf, eraw)


_TB = 1000


def _blk_body(agg_ref, inn_ref, ws_ref, bsum_ref, at_ref, w1_ref, b1_ref,
              w2_ref, z_ref, s2_ref):
    g = pl.program_id(0)
    w1 = w1_ref[...]
    b1 = b1_ref[...]
    w2r = w2_ref[...]
    sc = [agg_ref[r] * inn_ref[:, r][:, None] for r in range(_R)]
    for zi in range(10):
        tot = None
        for (r, wi) in _CONTRIB_IDX[zi]:
            y = jnp.dot(sc[r], ws_ref[wi], preferred_element_type=jnp.float32)
            tot = y if tot is None else tot + y
        tot = tot + bsum_ref[zi]
        z = jnp.where(tot >= 0, tot, tot * at_ref[zi])
        z_ref[zi] = z
        t = jnp.tanh(jnp.dot(z, w1, preferred_element_type=jnp.float32) + b1)
        sco = jnp.sum(t * w2r)
        srow = jnp.full((_D,), sco, jnp.float32)

        @pl.when(g == 0)
        def _():
            s2_ref[zi] = srow

        @pl.when(g > 0)
        def _():
            s2_ref[zi] = s2_ref[zi] + srow


def _run_blk(agg, inn, ws, bsum, at, w1, b1r, w2r):
    return pl.pallas_call(
        _blk_body,
        grid=(_N // _TB,),
        in_specs=[
            pl.BlockSpec((_R, _TB, _D), lambda g: (0, g, 0)),
            pl.BlockSpec((_TB, _R), lambda g: (g, 0)),
            pl.BlockSpec((15, _D, _D), lambda g: (0, 0, 0)),
            pl.BlockSpec((10, 1, _D), lambda g: (0, 0, 0)),
            pl.BlockSpec((10, 1, _D), lambda g: (0, 0, 0)),
            pl.BlockSpec((_D, _D), lambda g: (0, 0)),
            pl.BlockSpec((1, _D), lambda g: (0, 0)),
            pl.BlockSpec((1, _D), lambda g: (0, 0)),
        ],
        out_specs=[
            pl.BlockSpec((10, _TB, _D), lambda g: (0, g, 0)),
            pl.BlockSpec((10, _D), lambda g: (0, 0)),
        ],
        out_shape=[
            jax.ShapeDtypeStruct((10, _N, _D), jnp.float32),
            jax.ShapeDtypeStruct((10, _D), jnp.float32),
        ],
    )(agg, inn, ws, bsum, at, w1, b1r, w2r)


def _comb_body(z_ref, s2_ref, o0, o1, o2, o3, o4):
    outs = [o0, o1, o2, o3, o4]
    for nt in range(5):
        s0 = s2_ref[2 * nt]
        s1 = s2_ref[2 * nt + 1]
        m = jnp.maximum(s0, s1)
        e0 = jnp.exp((s0 - m) * (1.0 / _N))
        e1 = jnp.exp((s1 - m) * (1.0 / _N))
        b0 = e0 / (e0 + e1)
        b1 = e1 / (e0 + e1)
        outs[nt][...] = (z_ref[2 * nt] * b0[None, :]
                         + z_ref[2 * nt + 1] * b1[None, :])


def _run_comb(z, s2):
    return pl.pallas_call(
        _comb_body,
        grid=(_N // _TB,),
        in_specs=[
            pl.BlockSpec((10, _TB, _D), lambda g: (0, g, 0)),
            pl.BlockSpec((10, _D), lambda g: (0, 0)),
        ],
        out_specs=[pl.BlockSpec((_TB, _D), lambda g: (g, 0))] * 5,
        out_shape=[jax.ShapeDtypeStruct((_N, _D), jnp.float32)] * 5,
    )(z, s2)


def kernel(x_drug, x_disease, x_protein, x_gene, x_pathway, params,
           edge_drug_drug, edge_drug_disease, edge_disease_disease,
           edge_drug_protein, edge_protein_protein, edge_protein_gene,
           edge_gene_gene, edge_gene_pathway, edge_pathway_pathway,
           edge_pathway_disease):
    edges = [edge_drug_drug, edge_drug_disease, edge_disease_disease,
             edge_drug_protein, edge_protein_protein, edge_protein_gene,
             edge_gene_gene, edge_gene_pathway, edge_pathway_pathway,
             edge_pathway_disease]
    x5 = jnp.stack([x_drug, x_disease, x_protein, x_gene, x_pathway])
    eraw = jnp.concatenate([e.reshape(-1) for e in edges])

    degp = _run_deg(eraw)
    deg4 = degp.reshape(_R, 2, 16, _N)
    xs, inn = _run_norm(x5, deg4)
    innt = inn[:, 0, :].T
    xsf = xs.reshape(_R * _N, _D)
    agg = _run_agg(xsf, eraw)

    ws = jnp.stack([params['W_%s_%s' % (b, rl)] for (b, rl) in _WLIST])
    bsum = jnp.stack([
        sum(params['b_%s_%s' % (_ZBLOCK[zi], rl)] for (_, rl) in _CONTRIB[zi])
        for zi in range(10)]).reshape(10, 1, _D)
    at = jnp.stack([jnp.full((_D,), params['a_' + blk]) for blk in _ZBLOCK]
                   ).reshape(10, 1, _D)
    b1r = params['att_b1'].reshape(1, _D)
    w2r = params['att_w2'].reshape(1, _D)

    z, s2 = _run_blk(agg, innt, ws, bsum, at, params['att_W1'], b1r, w2r)
    outs = _run_comb(z, s2)
    return tuple(outs)


# speedup vs baseline: 1.0087x; 1.0087x over previous
import functools

import jax
import jax.numpy as jnp
from jax import lax
from jax.experimental import pallas as pl
from jax.experimental.pallas import tpu as pltpu
from jax.experimental.pallas import tpu_sc as plsc

_N = 10000
_D = 128
_E = 160000
_R = 10
_CH = 128
_NCH = _E // _CH
_SIDE = _R * _E

_RELS = ['drug_drug', 'drug_disease', 'disease_disease', 'drug_protein',
         'protein_protein', 'protein_gene', 'gene_gene', 'gene_pathway',
         'pathway_pathway', 'pathway_disease']
_SRC_T = [0, 0, 1, 0, 2, 2, 3, 3, 4, 4]

_ZBLOCK = ['dd', 'dp', 'dd', 'pd', 'dp', 'pg', 'pg', 'gp', 'gp', 'pd']
_CONTRIB = [
    [(0, 'drug_drug')],
    [(0, 'drug_drug')],
    [(1, 'drug_disease'), (2, 'disease_disease')],
    [(9, 'pathway_disease'), (2, 'disease_disease')],
    [(3, 'drug_protein'), (4, 'protein_protein')],
    [(4, 'protein_protein')],
    [(5, 'protein_gene'), (6, 'gene_gene')],
    [(6, 'gene_gene')],
    [(7, 'gene_pathway'), (8, 'pathway_pathway')],
    [(8, 'pathway_pathway')],
]
_WLIST = []
_CONTRIB_IDX = []
for _zi, _lst in enumerate(_CONTRIB):
    _idxs = []
    for (_r, _rl) in _lst:
        _idxs.append((_r, len(_WLIST)))
        _WLIST.append((_ZBLOCK[_zi], _rl))
    _CONTRIB_IDX.append(_idxs)


_CE = 2000


def _deg_body(eraw, degp, bins, idxb0, idxb1, sem_i0, sem_i1):
    idxb = [idxb0, idxb1]
    c = lax.axis_index("c")
    s = lax.axis_index("s")
    zero16 = jnp.zeros((16,), jnp.float32)
    ones16 = jnp.ones((16,), jnp.float32)
    sems = [sem_i0, sem_i1]

    @pl.loop(0, _R * _N // 16)
    def _zero(j):
        bins[pl.ds(j * 16, 16)] = zero16

    nch = _R * (_E // _CE) // 16

    def _issue(t, b):
        r = t // 5
        ch = s + 16 * (t % 5)
        off = (2 * r + c) * _E + ch * _CE
        pltpu.async_copy(eraw.at[pl.ds(off, _CE)], idxb[b], sems[b])

    _issue(0, 0)
    _issue(1, 1)

    @pl.loop(0, nch // 2)
    def _chunks(g):
        for b in range(2):
            t = g * 2 + b
            rn = (t // 5) * _N
            pltpu.make_async_copy(eraw.at[pl.ds(0, _CE)], idxb[b],
                                  sems[b]).wait()

            @pl.loop(0, _CE // 16, unroll=4)
            def _scat(j):
                v = idxb[b][pl.ds(j * 16, 16)] + rn
                plsc.addupdate_scatter(bins, [v], ones16)

            @pl.when(t + 2 < nch)
            def _():
                _issue(t + 2, b)

    w = c * 16 + s
    for r in range(_R):
        pltpu.sync_copy(bins.at[pl.ds(r * _N, _N)],
                        degp.at[pl.ds((r * 32 + w) * _N, _N)])


def _run_deg(ecat):
    mesh = plsc.VectorSubcoreMesh(core_axis_name="c", subcore_axis_name="s",
                                  num_cores=2, num_subcores=16)
    return pl.kernel(
        _deg_body,
        out_type=jax.ShapeDtypeStruct((32 * _R * _N,), jnp.float32),
        mesh=mesh,
        compiler_params=pltpu.CompilerParams(needs_layout_passes=False),
        scratch_types=[
            pltpu.VMEM((_R * _N,), jnp.float32),
            pltpu.VMEM((_CE,), jnp.int32),
            pltpu.VMEM((_CE,), jnp.int32),
            pltpu.SemaphoreType.DMA,
            pltpu.SemaphoreType.DMA,
        ],
    )(ecat)


def _norm_body(x0, x1, x2, x3, x4, deg_ref, xs_ref, inn_ref):
    r = pl.program_id(0)
    deg = deg_ref[...]
    outd = jnp.sum(deg[0, 0, :, :], axis=0)
    ind = jnp.sum(deg[0, 1, :, :], axis=0)
    onorm = lax.rsqrt(jnp.maximum(outd, 1.0))
    inorm = lax.rsqrt(jnp.maximum(ind, 1.0))
    st = _src_t_of(r)
    for k, xr in enumerate([x0, x1, x2, x3, x4]):
        @pl.when(st == k)
        def _():
            xs_ref[0] = xr[...] * onorm[:, None]
    inn_ref[0, 0] = inorm


def _src_t_of(r):
    return jnp.where(r >= 4, r // 2, jnp.where(r == 2, 1, 0))


def _run_norm(xlist, deg4):
    return pl.pallas_call(
        _norm_body,
        grid=(_R,),
        in_specs=[pl.BlockSpec((_N, _D), lambda r: (0, 0))] * 5 + [
            pl.BlockSpec((1, 2, 16, _N), lambda r: (r, 0, 0, 0)),
        ],
        out_specs=[
            pl.BlockSpec((1, _N, _D), lambda r: (r, 0, 0)),
            pl.BlockSpec((1, 1, _N), lambda r: (r, 0, 0)),
        ],
        out_shape=[
            jax.ShapeDtypeStruct((_R, _N, _D), jnp.float32),
            jax.ShapeDtypeStruct((_R, 1, _N), jnp.float32),
        ],
    )(*xlist, deg4)


def _agg_body(xsf, eraw, agg, acc, idx2, rows, zrow,
              sem_i0, sem_i1, sem_i2, sem_i3, sem_g0, sem_g1,
              sem_s0, sem_s1):
    c = lax.axis_index("c")
    s = lax.axis_index("s")
    zero16 = jnp.zeros((16,), jnp.float32)
    sems_i = [sem_i0, sem_i1, sem_i2, sem_i3]
    sems_g = [sem_g0, sem_g1]
    sems_s = [sem_s0, sem_s1]

    @pl.loop(0, 80)
    def _zz(j):
        for t in range(8):
            zrow[j, pl.ds(t * 16, 16)] = zero16

    def _zero_stripe():
        @pl.loop(0, 8)
        def _zs(t):
            g = s + 16 * t

            @pl.when(g < 125)
            def _():
                pltpu.sync_copy(zrow, acc.at[pl.ds(g * 80, 80)])

    _zero_stripe()
    for i in range(_R // 2):
        r = i * 2 + c
        plsc.subcore_barrier()

        rn = r * _N

        def _issue_idx(t, bi):
            @pl.when(s + 16 * t < _NCH)
            def _():
                off = 2 * r * _E + (s + 16 * t) * _CH
                pltpu.async_copy(eraw.at[pl.ds(off, _CH)], idx2.at[bi, 0],
                                 sems_i[bi])
                pltpu.async_copy(eraw.at[pl.ds(off + _E, _CH)],
                                 idx2.at[bi, 1], sems_i[bi])

        def _wait_adjust_idx(t, bi):
            @pl.when(s + 16 * t < _NCH)
            def _():
                pltpu.make_async_copy(eraw.at[pl.ds(0, _CH)],
                                      idx2.at[bi, 0], sems_i[bi]).wait()
                pltpu.make_async_copy(eraw.at[pl.ds(0, _CH)],
                                      idx2.at[bi, 1], sems_i[bi]).wait()
                for j in range(8):
                    sl = pl.ds(j * 16, 16)
                    idx2[bi, 0, sl] = idx2[bi, 0, sl] + rn

        def _issue_gather(t, bi, bg):
            @pl.when(s + 16 * t < _NCH)
            def _():
                pltpu.async_copy(xsf.at[idx2.at[bi, 0]], rows.at[bg],
                                 sems_g[bg])

        def _wait_gather(t, bg):
            @pl.when(s + 16 * t < _NCH)
            def _():
                pltpu.make_async_copy(xsf.at[pl.ds(0, _CH)], rows.at[bg],
                                      sems_g[bg]).wait()

        def _issue_scatter(t, bi, bg):
            @pl.when(s + 16 * t < _NCH)
            def _():
                pltpu.async_copy(rows.at[bg], acc.at[idx2.at[bi, 1]],
                                 sems_s[bg], add=True)

        def _wait_scatter(t, bg, extra_cond):
            @pl.when(jnp.logical_and(extra_cond, s + 16 * t < _NCH))
            def _():
                pltpu.make_async_copy(xsf.at[pl.ds(0, _CH)], rows.at[bg],
                                      sems_s[bg]).wait()

        _issue_idx(0, 0)
        _issue_idx(1, 1)
        _issue_idx(2, 2)
        _wait_adjust_idx(0, 0)
        _issue_gather(0, 0, 0)

        @pl.loop(0, 21)
        def _chunks(g):
            for u in range(4):
                t = g * 4 + u
                _wait_adjust_idx(t + 1, (u + 1) % 4)
                _wait_gather(t, u % 2)
                _wait_scatter(t - 1, (u + 1) % 2, t >= 1)
                _issue_gather(t + 1, (u + 1) % 4, (u + 1) % 2)
                _issue_scatter(t, u % 4, u % 2)
                _issue_idx(t + 3, (u + 3) % 4)

        plsc.subcore_barrier()

        @pl.loop(0, 8)
        def _drain(t):
            g = s + 16 * t

            @pl.when(g < 125)
            def _():
                sl = pl.ds(g * 80, 80)
                pltpu.sync_copy(acc.at[sl], agg.at[r, sl])

        if i < _R // 2 - 1:
            _zero_stripe()


def _run_agg(xsf, eraw):
    mesh = plsc.VectorSubcoreMesh(core_axis_name="c", subcore_axis_name="s",
                                  num_cores=2, num_subcores=16)
    return pl.kernel(
        _agg_body,
        out_type=jax.ShapeDtypeStruct((_R, _N, _D), jnp.float32),
        mesh=mesh,
        compiler_params=pltpu.CompilerParams(needs_layout_passes=False),
        scratch_types=[
            pltpu.VMEM_SHARED((_N, _D), jnp.float32),
            pltpu.VMEM((4, 2, _CH), jnp.int32),
            pltpu.VMEM((2, _CH, _D), jnp.float32),
            pltpu.VMEM((80, _D), jnp.float32),
        ] + [pltpu.SemaphoreType.DMA] * 8,
    )(xsf, eraw)


_TB = 1000


def _blk_body(agg_ref, inn_ref, ws_ref, bsum_ref, at_ref, w1_ref, b1_ref,
              w2_ref, z_ref, s2_ref):
    g = pl.program_id(0)
    w1 = w1_ref[...]
    b1 = b1_ref[...]
    w2r = w2_ref[...]
    sc = [agg_ref[r] * inn_ref[:, r][:, None] for r in range(_R)]
    for zi in range(10):
        tot = None
        for (r, wi) in _CONTRIB_IDX[zi]:
            y = jnp.dot(sc[r], ws_ref[wi], preferred_element_type=jnp.float32)
            tot = y if tot is None else tot + y
        tot = tot + bsum_ref[zi]
        z = jnp.where(tot >= 0, tot, tot * at_ref[zi])
        z_ref[zi] = z.astype(jnp.bfloat16)
        t = jnp.tanh(jnp.dot(z, w1, preferred_element_type=jnp.float32) + b1)
        sco = jnp.sum(t * w2r)
        srow = jnp.full((_D,), sco, jnp.float32)

        @pl.when(g == 0)
        def _():
            s2_ref[zi] = srow

        @pl.when(g > 0)
        def _():
            s2_ref[zi] = s2_ref[zi] + srow


def _run_blk(agg, inn, ws, bsum, at, w1, b1r, w2r):
    return pl.pallas_call(
        _blk_body,
        grid=(_N // _TB,),
        in_specs=[
            pl.BlockSpec((_R, _TB, _D), lambda g: (0, g, 0)),
            pl.BlockSpec((_TB, _R), lambda g: (g, 0)),
            pl.BlockSpec((15, _D, _D), lambda g: (0, 0, 0)),
            pl.BlockSpec((10, 1, _D), lambda g: (0, 0, 0)),
            pl.BlockSpec((10, 1, _D), lambda g: (0, 0, 0)),
            pl.BlockSpec((_D, _D), lambda g: (0, 0)),
            pl.BlockSpec((1, _D), lambda g: (0, 0)),
            pl.BlockSpec((1, _D), lambda g: (0, 0)),
        ],
        out_specs=[
            pl.BlockSpec((10, _TB, _D), lambda g: (0, g, 0)),
            pl.BlockSpec((10, _D), lambda g: (0, 0)),
        ],
        out_shape=[
            jax.ShapeDtypeStruct((10, _N, _D), jnp.bfloat16),
            jax.ShapeDtypeStruct((10, _D), jnp.float32),
        ],
    )(agg, inn, ws, bsum, at, w1, b1r, w2r)


def _comb_body(z_ref, s2_ref, o0, o1, o2, o3, o4):
    outs = [o0, o1, o2, o3, o4]
    for nt in range(5):
        s0 = s2_ref[2 * nt]
        s1 = s2_ref[2 * nt + 1]
        m = jnp.maximum(s0, s1)
        e0 = jnp.exp((s0 - m) * (1.0 / _N))
        e1 = jnp.exp((s1 - m) * (1.0 / _N))
        b0 = e0 / (e0 + e1)
        b1 = e1 / (e0 + e1)
        outs[nt][...] = (z_ref[2 * nt].astype(jnp.float32) * b0[None, :]
                         + z_ref[2 * nt + 1].astype(jnp.float32) * b1[None, :])


def _run_comb(z, s2):
    return pl.pallas_call(
        _comb_body,
        grid=(_N // _TB,),
        in_specs=[
            pl.BlockSpec((10, _TB, _D), lambda g: (0, g, 0)),
            pl.BlockSpec((10, _D), lambda g: (0, 0)),
        ],
        out_specs=[pl.BlockSpec((_TB, _D), lambda g: (g, 0))] * 5,
        out_shape=[jax.ShapeDtypeStruct((_N, _D), jnp.float32)] * 5,
    )(z, s2)


def kernel(x_drug, x_disease, x_protein, x_gene, x_pathway, params,
           edge_drug_drug, edge_drug_disease, edge_disease_disease,
           edge_drug_protein, edge_protein_protein, edge_protein_gene,
           edge_gene_gene, edge_gene_pathway, edge_pathway_pathway,
           edge_pathway_disease):
    edges = [edge_drug_drug, edge_drug_disease, edge_disease_disease,
             edge_drug_protein, edge_protein_protein, edge_protein_gene,
             edge_gene_gene, edge_gene_pathway, edge_pathway_pathway,
             edge_pathway_disease]
    xlist = [x_drug, x_disease, x_protein, x_gene, x_pathway]
    eraw = jnp.concatenate([e.reshape(-1) for e in edges])

    degp = _run_deg(eraw)
    deg4 = degp.reshape(_R, 2, 16, _N)
    xs, inn = _run_norm(xlist, deg4)
    innt = inn[:, 0, :].T
    xsf = xs.reshape(_R * _N, _D)
    agg = _run_agg(xsf, eraw)

    ws = jnp.stack([params['W_%s_%s' % (b, rl)] for (b, rl) in _WLIST])
    bsum = jnp.stack([
        sum(params['b_%s_%s' % (_ZBLOCK[zi], rl)] for (_, rl) in _CONTRIB[zi])
        for zi in range(10)]).reshape(10, 1, _D)
    at = jnp.stack([jnp.full((_D,), params['a_' + blk]) for blk in _ZBLOCK]
                   ).reshape(10, 1, _D)
    b1r = params['att_b1'].reshape(1, _D)
    w2r = params['att_w2'].reshape(1, _D)

    z, s2 = _run_blk(agg, innt, ws, bsum, at, params['att_W1'], b1r, w2r)
    outs = _run_comb(z, s2)
    return tuple(outs)


# speedup vs baseline: 1.0447x; 1.0357x over previous
import functools

import jax
import jax.numpy as jnp
from jax import lax
from jax.experimental import pallas as pl
from jax.experimental.pallas import tpu as pltpu
from jax.experimental.pallas import tpu_sc as plsc

_N = 10000
_D = 128
_E = 160000
_R = 10
_CH = 128
_NCH = _E // _CH
_SIDE = _R * _E

_RELS = ['drug_drug', 'drug_disease', 'disease_disease', 'drug_protein',
         'protein_protein', 'protein_gene', 'gene_gene', 'gene_pathway',
         'pathway_pathway', 'pathway_disease']
_SRC_T = [0, 0, 1, 0, 2, 2, 3, 3, 4, 4]

_ZBLOCK = ['dd', 'dp', 'dd', 'pd', 'dp', 'pg', 'pg', 'gp', 'gp', 'pd']
_CONTRIB = [
    [(0, 'drug_drug')],
    [(0, 'drug_drug')],
    [(1, 'drug_disease'), (2, 'disease_disease')],
    [(9, 'pathway_disease'), (2, 'disease_disease')],
    [(3, 'drug_protein'), (4, 'protein_protein')],
    [(4, 'protein_protein')],
    [(5, 'protein_gene'), (6, 'gene_gene')],
    [(6, 'gene_gene')],
    [(7, 'gene_pathway'), (8, 'pathway_pathway')],
    [(8, 'pathway_pathway')],
]
_WLIST = []
_CONTRIB_IDX = []
for _zi, _lst in enumerate(_CONTRIB):
    _idxs = []
    for (_r, _rl) in _lst:
        _idxs.append((_r, len(_WLIST)))
        _WLIST.append((_ZBLOCK[_zi], _rl))
    _CONTRIB_IDX.append(_idxs)


_CE = 2000


def _deg_body(eraw, degp, bins, idxb0, idxb1, sem_i0, sem_i1):
    idxb = [idxb0, idxb1]
    c = lax.axis_index("c")
    s = lax.axis_index("s")
    zero16 = jnp.zeros((16,), jnp.float32)
    ones16 = jnp.ones((16,), jnp.float32)
    sems = [sem_i0, sem_i1]

    @pl.loop(0, _R * _N // 16)
    def _zero(j):
        bins[pl.ds(j * 16, 16)] = zero16

    nch = _R * (_E // _CE) // 16

    def _issue(t, b):
        r = t // 5
        ch = s + 16 * (t % 5)
        off = (2 * r + c) * _E + ch * _CE
        pltpu.async_copy(eraw.at[pl.ds(off, _CE)], idxb[b], sems[b])

    _issue(0, 0)
    _issue(1, 1)

    @pl.loop(0, nch // 2)
    def _chunks(g):
        for b in range(2):
            t = g * 2 + b
            rn = (t // 5) * _N
            pltpu.make_async_copy(eraw.at[pl.ds(0, _CE)], idxb[b],
                                  sems[b]).wait()

            @pl.loop(0, _CE // 16, unroll=4)
            def _scat(j):
                v = idxb[b][pl.ds(j * 16, 16)] + rn
                plsc.addupdate_scatter(bins, [v], ones16)

            @pl.when(t + 2 < nch)
            def _():
                _issue(t + 2, b)

    w = c * 16 + s
    for r in range(_R):
        pltpu.sync_copy(bins.at[pl.ds(r * _N, _N)],
                        degp.at[pl.ds((r * 32 + w) * _N, _N)])


def _run_deg(ecat):
    mesh = plsc.VectorSubcoreMesh(core_axis_name="c", subcore_axis_name="s",
                                  num_cores=2, num_subcores=16)
    return pl.kernel(
        _deg_body,
        out_type=jax.ShapeDtypeStruct((32 * _R * _N,), jnp.float32),
        mesh=mesh,
        compiler_params=pltpu.CompilerParams(needs_layout_passes=False),
        scratch_types=[
            pltpu.VMEM((_R * _N,), jnp.float32),
            pltpu.VMEM((_CE,), jnp.int32),
            pltpu.VMEM((_CE,), jnp.int32),
            pltpu.SemaphoreType.DMA,
            pltpu.SemaphoreType.DMA,
        ],
    )(ecat)


def _norm_body(x0, x1, x2, x3, x4, deg_ref, xs_ref, inn_ref):
    r = pl.program_id(0)
    deg = deg_ref[...]
    outd = jnp.sum(deg[0, 0, :, :], axis=0)
    ind = jnp.sum(deg[0, 1, :, :], axis=0)
    onorm = lax.rsqrt(jnp.maximum(outd, 1.0))
    inorm = lax.rsqrt(jnp.maximum(ind, 1.0))
    st = _src_t_of(r)
    for k, xr in enumerate([x0, x1, x2, x3, x4]):
        @pl.when(st == k)
        def _():
            xs_ref[0] = xr[...] * onorm[:, None]
    inn_ref[0, 0] = inorm


def _src_t_of(r):
    return jnp.where(r >= 4, r // 2, jnp.where(r == 2, 1, 0))


def _run_norm(xlist, deg4):
    return pl.pallas_call(
        _norm_body,
        grid=(_R,),
        in_specs=[pl.BlockSpec((_N, _D), lambda r: (0, 0))] * 5 + [
            pl.BlockSpec((1, 2, 16, _N), lambda r: (r, 0, 0, 0)),
        ],
        out_specs=[
            pl.BlockSpec((1, _N, _D), lambda r: (r, 0, 0)),
            pl.BlockSpec((1, 1, _N), lambda r: (r, 0, 0)),
        ],
        out_shape=[
            jax.ShapeDtypeStruct((_R, _N, _D), jnp.float32),
            jax.ShapeDtypeStruct((_R, 1, _N), jnp.float32),
        ],
    )(*xlist, deg4)


def _agg_body(rels, xsf, eraw, agg, acc, idx2, rows, zrow,
              sem_i0, sem_i1, sem_i2, sem_i3, sem_g0, sem_g1,
              sem_s0, sem_s1):
    c = lax.axis_index("c")
    s = lax.axis_index("s")
    zero16 = jnp.zeros((16,), jnp.float32)
    sems_i = [sem_i0, sem_i1, sem_i2, sem_i3]
    sems_g = [sem_g0, sem_g1]
    sems_s = [sem_s0, sem_s1]

    @pl.loop(0, 80)
    def _zz(j):
        for t in range(8):
            zrow[j, pl.ds(t * 16, 16)] = zero16

    def _zero_stripe():
        @pl.loop(0, 8)
        def _zs(t):
            g = s + 16 * t

            @pl.when(g < 125)
            def _():
                pltpu.sync_copy(zrow, acc.at[pl.ds(g * 80, 80)])

    _zero_stripe()
    for i in range(len(rels) // 2):
        p0, p1 = rels[2 * i], rels[2 * i + 1]
        r = p0 + c * (p1 - p0)
        li = 2 * i + c
        plsc.subcore_barrier()

        rn = r * _N

        def _issue_idx(t, bi):
            @pl.when(s + 16 * t < _NCH)
            def _():
                off = 2 * r * _E + (s + 16 * t) * _CH
                pltpu.async_copy(eraw.at[pl.ds(off, _CH)], idx2.at[bi, 0],
                                 sems_i[bi])
                pltpu.async_copy(eraw.at[pl.ds(off + _E, _CH)],
                                 idx2.at[bi, 1], sems_i[bi])

        def _wait_adjust_idx(t, bi):
            @pl.when(s + 16 * t < _NCH)
            def _():
                pltpu.make_async_copy(eraw.at[pl.ds(0, _CH)],
                                      idx2.at[bi, 0], sems_i[bi]).wait()
                pltpu.make_async_copy(eraw.at[pl.ds(0, _CH)],
                                      idx2.at[bi, 1], sems_i[bi]).wait()
                for j in range(8):
                    sl = pl.ds(j * 16, 16)
                    idx2[bi, 0, sl] = idx2[bi, 0, sl] + rn

        def _issue_gather(t, bi, bg):
            @pl.when(s + 16 * t < _NCH)
            def _():
                pltpu.async_copy(xsf.at[idx2.at[bi, 0]], rows.at[bg],
                                 sems_g[bg])

        def _wait_gather(t, bg):
            @pl.when(s + 16 * t < _NCH)
            def _():
                pltpu.make_async_copy(xsf.at[pl.ds(0, _CH)], rows.at[bg],
                                      sems_g[bg]).wait()

        def _issue_scatter(t, bi, bg):
            @pl.when(s + 16 * t < _NCH)
            def _():
                pltpu.async_copy(rows.at[bg], acc.at[idx2.at[bi, 1]],
                                 sems_s[bg], add=True)

        def _wait_scatter(t, bg, extra_cond):
            @pl.when(jnp.logical_and(extra_cond, s + 16 * t < _NCH))
            def _():
                pltpu.make_async_copy(xsf.at[pl.ds(0, _CH)], rows.at[bg],
                                      sems_s[bg]).wait()

        _issue_idx(0, 0)
        _issue_idx(1, 1)
        _issue_idx(2, 2)
        _wait_adjust_idx(0, 0)
        _issue_gather(0, 0, 0)

        @pl.loop(0, 21)
        def _chunks(g):
            for u in range(4):
                t = g * 4 + u
                _wait_adjust_idx(t + 1, (u + 1) % 4)
                _wait_gather(t, u % 2)
                _wait_scatter(t - 1, (u + 1) % 2, t >= 1)
                _issue_gather(t + 1, (u + 1) % 4, (u + 1) % 2)
                _issue_scatter(t, u % 4, u % 2)
                _issue_idx(t + 3, (u + 3) % 4)

        plsc.subcore_barrier()

        @pl.loop(0, 8)
        def _drain(t):
            g = s + 16 * t

            @pl.when(g < 125)
            def _():
                sl = pl.ds(g * 80, 80)
                pltpu.sync_copy(acc.at[sl], agg.at[li, sl])

        if i < len(rels) // 2 - 1:
            _zero_stripe()


def _run_agg(xsf, eraw, rels):
    mesh = plsc.VectorSubcoreMesh(core_axis_name="c", subcore_axis_name="s",
                                  num_cores=2, num_subcores=16)
    return pl.kernel(
        functools.partial(_agg_body, rels),
        out_type=jax.ShapeDtypeStruct((len(rels), _N, _D), jnp.float32),
        mesh=mesh,
        compiler_params=pltpu.CompilerParams(needs_layout_passes=False),
        scratch_types=[
            pltpu.VMEM_SHARED((_N, _D), jnp.float32),
            pltpu.VMEM((4, 2, _CH), jnp.int32),
            pltpu.VMEM((2, _CH, _D), jnp.float32),
            pltpu.VMEM((80, _D), jnp.float32),
        ] + [pltpu.SemaphoreType.DMA] * 8,
    )(xsf, eraw)


_TB = 1000

_SPLIT_A = [0, 1, 2, 3, 4, 5]
_SPLIT_B = [6, 7, 8, 9]
_ZIS_A = [0, 1, 2, 4, 5]
_ZIS_B = [3, 6, 7, 8, 9]


def _loc(r):
    if r in _SPLIT_A:
        return 0, _SPLIT_A.index(r)
    return 1, _SPLIT_B.index(r)


def _contrib_spec(zis):
    spec, nw = [], 0
    for zi in zis:
        ent = []
        for (r, rl) in _CONTRIB[zi]:
            a, lr = _loc(r)
            ent.append((a, lr, r, nw, _ZBLOCK[zi], rl))
            nw += 1
        spec.append(ent)
    return spec, nw


def _blk_body(zis, spec, n_agg, *refs):
    agg_refs = refs[:n_agg]
    inn_ref, ws_ref, bsum_ref, at_ref, w1_ref, b1_ref, w2_ref = \
        refs[n_agg:n_agg + 7]
    z_ref, s2_ref = refs[n_agg + 7:]
    g = pl.program_id(0)
    w1 = w1_ref[...]
    b1 = b1_ref[...]
    w2r = w2_ref[...]
    sc = {}
    for k in range(len(zis)):
        tot = None
        for (a, lr, gr, wi, _, _) in spec[k]:
            if (a, lr) not in sc:
                sc[(a, lr)] = agg_refs[a][lr] * inn_ref[:, gr][:, None]
            y = jnp.dot(sc[(a, lr)], ws_ref[wi],
                        preferred_element_type=jnp.float32)
            tot = y if tot is None else tot + y
        tot = tot + bsum_ref[k]
        z = jnp.where(tot >= 0, tot, tot * at_ref[k])
        z_ref[k] = z.astype(jnp.bfloat16)
        t = jnp.tanh(jnp.dot(z, w1, preferred_element_type=jnp.float32) + b1)
        sco = jnp.sum(t * w2r)
        srow = jnp.full((_D,), sco, jnp.float32)

        @pl.when(g == 0)
        def _():
            s2_ref[k] = srow

        @pl.when(g > 0)
        def _():
            s2_ref[k] = s2_ref[k] + srow


def _run_blk(zis, aggs, inn, params, w1, b1r, w2r):
    spec, nw = _contrib_spec(zis)
    ws = jnp.stack([params['W_%s_%s' % (blk, rl)]
                    for ent in spec for (_, _, _, _, blk, rl) in ent])
    bsum = jnp.stack([
        sum(params['b_%s_%s' % (_ZBLOCK[zi], rl)] for (_, rl) in _CONTRIB[zi])
        for zi in zis]).reshape(len(zis), 1, _D)
    at = jnp.stack([jnp.full((_D,), params['a_' + _ZBLOCK[zi]])
                    for zi in zis]).reshape(len(zis), 1, _D)
    nz = len(zis)
    body = functools.partial(_blk_body, zis, spec, len(aggs))
    return pl.pallas_call(
        body,
        grid=(_N // _TB,),
        in_specs=[
            pl.BlockSpec((a.shape[0], _TB, _D), lambda g: (0, g, 0))
            for a in aggs] + [
            pl.BlockSpec((_TB, _R), lambda g: (g, 0)),
            pl.BlockSpec((nw, _D, _D), lambda g: (0, 0, 0)),
            pl.BlockSpec((nz, 1, _D), lambda g: (0, 0, 0)),
            pl.BlockSpec((nz, 1, _D), lambda g: (0, 0, 0)),
            pl.BlockSpec((_D, _D), lambda g: (0, 0)),
            pl.BlockSpec((1, _D), lambda g: (0, 0)),
            pl.BlockSpec((1, _D), lambda g: (0, 0)),
        ],
        out_specs=[
            pl.BlockSpec((nz, _TB, _D), lambda g: (0, g, 0)),
            pl.BlockSpec((nz, _D), lambda g: (0, 0)),
        ],
        out_shape=[
            jax.ShapeDtypeStruct((nz, _N, _D), jnp.bfloat16),
            jax.ShapeDtypeStruct((nz, _D), jnp.float32),
        ],
    )(*aggs, inn, ws, bsum, at, w1, b1r, w2r)


_PAIRS = [((0, 0), (0, 1)), ((0, 2), (1, 0)), ((0, 3), (0, 4)),
          ((1, 1), (1, 2)), ((1, 3), (1, 4))]


def _comb_body(za_ref, zb_ref, sa_ref, sb_ref, o0, o1, o2, o3, o4):
    outs = [o0, o1, o2, o3, o4]
    zs = [za_ref, zb_ref]
    ss = [sa_ref, sb_ref]
    for nt, ((a0, i0), (a1, i1)) in enumerate(_PAIRS):
        s0 = ss[a0][i0]
        s1 = ss[a1][i1]
        m = jnp.maximum(s0, s1)
        e0 = jnp.exp((s0 - m) * (1.0 / _N))
        e1 = jnp.exp((s1 - m) * (1.0 / _N))
        b0 = e0 / (e0 + e1)
        b1 = e1 / (e0 + e1)
        outs[nt][...] = (zs[a0][i0].astype(jnp.float32) * b0[None, :]
                         + zs[a1][i1].astype(jnp.float32) * b1[None, :])


def _run_comb(za, zb, sa, sb):
    return pl.pallas_call(
        _comb_body,
        grid=(_N // _TB,),
        in_specs=[
            pl.BlockSpec((5, _TB, _D), lambda g: (0, g, 0)),
            pl.BlockSpec((5, _TB, _D), lambda g: (0, g, 0)),
            pl.BlockSpec((5, _D), lambda g: (0, 0)),
            pl.BlockSpec((5, _D), lambda g: (0, 0)),
        ],
        out_specs=[pl.BlockSpec((_TB, _D), lambda g: (g, 0))] * 5,
        out_shape=[jax.ShapeDtypeStruct((_N, _D), jnp.float32)] * 5,
    )(za, zb, sa, sb)


def kernel(x_drug, x_disease, x_protein, x_gene, x_pathway, params,
           edge_drug_drug, edge_drug_disease, edge_disease_disease,
           edge_drug_protein, edge_protein_protein, edge_protein_gene,
           edge_gene_gene, edge_gene_pathway, edge_pathway_pathway,
           edge_pathway_disease):
    edges = [edge_drug_drug, edge_drug_disease, edge_disease_disease,
             edge_drug_protein, edge_protein_protein, edge_protein_gene,
             edge_gene_gene, edge_gene_pathway, edge_pathway_pathway,
             edge_pathway_disease]
    xlist = [x_drug, x_disease, x_protein, x_gene, x_pathway]
    eraw = jnp.concatenate([e.reshape(-1) for e in edges])

    degp = _run_deg(eraw)
    deg4 = degp.reshape(_R, 2, 16, _N)
    xs, inn = _run_norm(xlist, deg4)
    innt = inn[:, 0, :].T
    xsf = xs.reshape(_R * _N, _D)
    agg_a = _run_agg(xsf, eraw, _SPLIT_A)
    agg_b = _run_agg(xsf, eraw, _SPLIT_B)

    b1r = params['att_b1'].reshape(1, _D)
    w2r = params['att_w2'].reshape(1, _D)
    w1 = params['att_W1']

    za, sa = _run_blk(_ZIS_A, [agg_a], innt, params, w1, b1r, w2r)
    zb, sb = _run_blk(_ZIS_B, [agg_a, agg_b], innt, params, w1, b1r, w2r)
    outs = _run_comb(za, zb, sa, sb)
    return tuple(outs)


# speedup vs baseline: 1.0448x; 1.0001x over previous
import functools

import jax
import jax.numpy as jnp
from jax import lax
from jax.experimental import pallas as pl
from jax.experimental.pallas import tpu as pltpu
from jax.experimental.pallas import tpu_sc as plsc

_N = 10000
_D = 128
_E = 160000
_R = 10
_CH = 128
_NCH = _E // _CH
_SIDE = _R * _E

_RELS = ['drug_drug', 'drug_disease', 'disease_disease', 'drug_protein',
         'protein_protein', 'protein_gene', 'gene_gene', 'gene_pathway',
         'pathway_pathway', 'pathway_disease']
_SRC_T = [0, 0, 1, 0, 2, 2, 3, 3, 4, 4]

_ZBLOCK = ['dd', 'dp', 'dd', 'pd', 'dp', 'pg', 'pg', 'gp', 'gp', 'pd']
_CONTRIB = [
    [(0, 'drug_drug')],
    [(0, 'drug_drug')],
    [(1, 'drug_disease'), (2, 'disease_disease')],
    [(9, 'pathway_disease'), (2, 'disease_disease')],
    [(3, 'drug_protein'), (4, 'protein_protein')],
    [(4, 'protein_protein')],
    [(5, 'protein_gene'), (6, 'gene_gene')],
    [(6, 'gene_gene')],
    [(7, 'gene_pathway'), (8, 'pathway_pathway')],
    [(8, 'pathway_pathway')],
]
_WLIST = []
_CONTRIB_IDX = []
for _zi, _lst in enumerate(_CONTRIB):
    _idxs = []
    for (_r, _rl) in _lst:
        _idxs.append((_r, len(_WLIST)))
        _WLIST.append((_ZBLOCK[_zi], _rl))
    _CONTRIB_IDX.append(_idxs)


_CE = 2000


def _deg_body(eraw, degp, bins, idxb0, idxb1, sem_i0, sem_i1):
    idxb = [idxb0, idxb1]
    c = lax.axis_index("c")
    s = lax.axis_index("s")
    zero16 = jnp.zeros((16,), jnp.float32)
    ones16 = jnp.ones((16,), jnp.float32)
    sems = [sem_i0, sem_i1]

    @pl.loop(0, _R * _N // 16)
    def _zero(j):
        bins[pl.ds(j * 16, 16)] = zero16

    nch = _R * (_E // _CE) // 16

    def _issue(t, b):
        r = t // 5
        ch = s + 16 * (t % 5)
        off = (2 * r + c) * _E + ch * _CE
        pltpu.async_copy(eraw.at[pl.ds(off, _CE)], idxb[b], sems[b])

    _issue(0, 0)
    _issue(1, 1)

    @pl.loop(0, nch // 2)
    def _chunks(g):
        for b in range(2):
            t = g * 2 + b
            rn = (t // 5) * _N
            pltpu.make_async_copy(eraw.at[pl.ds(0, _CE)], idxb[b],
                                  sems[b]).wait()

            @pl.loop(0, _CE // 16, unroll=4)
            def _scat(j):
                v = idxb[b][pl.ds(j * 16, 16)] + rn
                plsc.addupdate_scatter(bins, [v], ones16)

            @pl.when(t + 2 < nch)
            def _():
                _issue(t + 2, b)

    w = c * 16 + s
    for r in range(_R):
        pltpu.sync_copy(bins.at[pl.ds(r * _N, _N)],
                        degp.at[pl.ds((r * 32 + w) * _N, _N)])


def _run_deg(ecat):
    mesh = plsc.VectorSubcoreMesh(core_axis_name="c", subcore_axis_name="s",
                                  num_cores=2, num_subcores=16)
    return pl.kernel(
        _deg_body,
        out_type=jax.ShapeDtypeStruct((32 * _R * _N,), jnp.float32),
        mesh=mesh,
        compiler_params=pltpu.CompilerParams(needs_layout_passes=False),
        scratch_types=[
            pltpu.VMEM((_R * _N,), jnp.float32),
            pltpu.VMEM((_CE,), jnp.int32),
            pltpu.VMEM((_CE,), jnp.int32),
            pltpu.SemaphoreType.DMA,
            pltpu.SemaphoreType.DMA,
        ],
    )(ecat)


def _norm_body(r0, x0, x1, x2, x3, x4, deg_ref, xs_ref, inn_ref):
    r = pl.program_id(0) + r0
    deg = deg_ref[...]
    outd = jnp.sum(deg[0, 0, :, :], axis=0)
    ind = jnp.sum(deg[0, 1, :, :], axis=0)
    onorm = lax.rsqrt(jnp.maximum(outd, 1.0))
    inorm = lax.rsqrt(jnp.maximum(ind, 1.0))
    st = _src_t_of(r)
    for k, xr in enumerate([x0, x1, x2, x3, x4]):
        @pl.when(st == k)
        def _():
            xs_ref[0] = xr[...] * onorm[:, None]
    inn_ref[0, 0] = inorm


def _src_t_of(r):
    return jnp.where(r >= 4, r // 2, jnp.where(r == 2, 1, 0))


def _run_norm(xlist, deg4, r0, nr):
    return pl.pallas_call(
        functools.partial(_norm_body, r0),
        grid=(nr,),
        in_specs=[pl.BlockSpec((_N, _D), lambda r: (0, 0))] * 5 + [
            pl.BlockSpec((1, 2, 16, _N), lambda r: (r + r0, 0, 0, 0)),
        ],
        out_specs=[
            pl.BlockSpec((1, _N, _D), lambda r: (r, 0, 0)),
            pl.BlockSpec((1, 1, _N), lambda r: (r, 0, 0)),
        ],
        out_shape=[
            jax.ShapeDtypeStruct((nr, _N, _D), jnp.float32),
            jax.ShapeDtypeStruct((nr, 1, _N), jnp.float32),
        ],
    )(*xlist, deg4)


def _agg_body(rels, xsf, eraw, agg, acc, idx2, rows, zrow,
              sem_i0, sem_i1, sem_i2, sem_i3, sem_g0, sem_g1,
              sem_s0, sem_s1):
    c = lax.axis_index("c")
    s = lax.axis_index("s")
    zero16 = jnp.zeros((16,), jnp.float32)
    sems_i = [sem_i0, sem_i1, sem_i2, sem_i3]
    sems_g = [sem_g0, sem_g1]
    sems_s = [sem_s0, sem_s1]

    @pl.loop(0, 80)
    def _zz(j):
        for t in range(8):
            zrow[j, pl.ds(t * 16, 16)] = zero16

    def _zero_stripe():
        @pl.loop(0, 8)
        def _zs(t):
            g = s + 16 * t

            @pl.when(g < 125)
            def _():
                pltpu.sync_copy(zrow, acc.at[pl.ds(g * 80, 80)])

    _zero_stripe()
    for i in range(len(rels) // 2):
        p0, p1 = rels[2 * i], rels[2 * i + 1]
        r = p0 + c * (p1 - p0)
        li = 2 * i + c
        plsc.subcore_barrier()

        rn = (r - rels[0]) * _N

        def _issue_idx(t, bi):
            @pl.when(s + 16 * t < _NCH)
            def _():
                off = 2 * r * _E + (s + 16 * t) * _CH
                pltpu.async_copy(eraw.at[pl.ds(off, _CH)], idx2.at[bi, 0],
                                 sems_i[bi])
                pltpu.async_copy(eraw.at[pl.ds(off + _E, _CH)],
                                 idx2.at[bi, 1], sems_i[bi])

        def _wait_adjust_idx(t, bi):
            @pl.when(s + 16 * t < _NCH)
            def _():
                pltpu.make_async_copy(eraw.at[pl.ds(0, _CH)],
                                      idx2.at[bi, 0], sems_i[bi]).wait()
                pltpu.make_async_copy(eraw.at[pl.ds(0, _CH)],
                                      idx2.at[bi, 1], sems_i[bi]).wait()
                for j in range(8):
                    sl = pl.ds(j * 16, 16)
                    idx2[bi, 0, sl] = idx2[bi, 0, sl] + rn

        def _issue_gather(t, bi, bg):
            @pl.when(s + 16 * t < _NCH)
            def _():
                pltpu.async_copy(xsf.at[idx2.at[bi, 0]], rows.at[bg],
                                 sems_g[bg])

        def _wait_gather(t, bg):
            @pl.when(s + 16 * t < _NCH)
            def _():
                pltpu.make_async_copy(xsf.at[pl.ds(0, _CH)], rows.at[bg],
                                      sems_g[bg]).wait()

        def _issue_scatter(t, bi, bg):
            @pl.when(s + 16 * t < _NCH)
            def _():
                pltpu.async_copy(rows.at[bg], acc.at[idx2.at[bi, 1]],
                                 sems_s[bg], add=True)

        def _wait_scatter(t, bg, extra_cond):
            @pl.when(jnp.logical_and(extra_cond, s + 16 * t < _NCH))
            def _():
                pltpu.make_async_copy(xsf.at[pl.ds(0, _CH)], rows.at[bg],
                                      sems_s[bg]).wait()

        _issue_idx(0, 0)
        _issue_idx(1, 1)
        _issue_idx(2, 2)
        _wait_adjust_idx(0, 0)
        _issue_gather(0, 0, 0)

        @pl.loop(0, 21)
        def _chunks(g):
            for u in range(4):
                t = g * 4 + u
                _wait_adjust_idx(t + 1, (u + 1) % 4)
                _wait_gather(t, u % 2)
                _wait_scatter(t - 1, (u + 1) % 2, t >= 1)
                _issue_gather(t + 1, (u + 1) % 4, (u + 1) % 2)
                _issue_scatter(t, u % 4, u % 2)
                _issue_idx(t + 3, (u + 3) % 4)

        plsc.subcore_barrier()

        @pl.loop(0, 8)
        def _drain(t):
            g = s + 16 * t

            @pl.when(g < 125)
            def _():
                sl = pl.ds(g * 80, 80)
                pltpu.sync_copy(acc.at[sl], agg.at[li, sl])

        if i < len(rels) // 2 - 1:
            _zero_stripe()


def _run_agg(xsf, eraw, rels):
    mesh = plsc.VectorSubcoreMesh(core_axis_name="c", subcore_axis_name="s",
                                  num_cores=2, num_subcores=16)
    return pl.kernel(
        functools.partial(_agg_body, rels),
        out_type=jax.ShapeDtypeStruct((len(rels), _N, _D), jnp.float32),
        mesh=mesh,
        compiler_params=pltpu.CompilerParams(needs_layout_passes=False),
        scratch_types=[
            pltpu.VMEM_SHARED((_N, _D), jnp.float32),
            pltpu.VMEM((4, 2, _CH), jnp.int32),
            pltpu.VMEM((2, _CH, _D), jnp.float32),
            pltpu.VMEM((80, _D), jnp.float32),
        ] + [pltpu.SemaphoreType.DMA] * 8,
    )(xsf, eraw)


_TB = 1000

_PART = [[0, 1], [2, 3, 4, 5], [6, 7, 8, 9]]


def _part_of(r):
    for pi, p in enumerate(_PART):
        if r in p:
            return pi
    raise AssertionError


_ZSTAGE = [max(_part_of(r) for (r, _) in _CONTRIB[zi]) for zi in range(10)]
_ZIS = [[zi for zi in range(10) if _ZSTAGE[zi] == k]
        for k in range(len(_PART))]
_STAGE_PARTS = [sorted({_part_of(r) for zi in _ZIS[k]
                        for (r, _) in _CONTRIB[zi]})
                for k in range(len(_PART))]


def _contrib_spec(k):
    parts = _STAGE_PARTS[k]
    spec, nw = [], 0
    for zi in _ZIS[k]:
        ent = []
        for (r, rl) in _CONTRIB[zi]:
            p = _part_of(r)
            ent.append((parts.index(p), _PART[p].index(r), r, nw,
                        _ZBLOCK[zi], rl))
            nw += 1
        spec.append(ent)
    return spec, nw


def _blk_body(zis, spec, n_agg, *refs):
    agg_refs = refs[:n_agg]
    inn_ref, ws_ref, bsum_ref, at_ref, w1_ref, b1_ref, w2_ref = \
        refs[n_agg:n_agg + 7]
    z_ref, s2_ref = refs[n_agg + 7:]
    g = pl.program_id(0)
    w1 = w1_ref[...]
    b1 = b1_ref[...]
    w2r = w2_ref[...]
    sc = {}
    for k in range(len(zis)):
        tot = None
        for (a, lr, gr, wi, _, _) in spec[k]:
            if (a, lr) not in sc:
                sc[(a, lr)] = agg_refs[a][lr] * inn_ref[:, gr][:, None]
            y = jnp.dot(sc[(a, lr)], ws_ref[wi],
                        preferred_element_type=jnp.float32)
            tot = y if tot is None else tot + y
        tot = tot + bsum_ref[k]
        z = jnp.where(tot >= 0, tot, tot * at_ref[k])
        z_ref[k] = z.astype(jnp.bfloat16)
        t = jnp.tanh(jnp.dot(z, w1, preferred_element_type=jnp.float32) + b1)
        sco = jnp.sum(t * w2r)
        srow = jnp.full((_D,), sco, jnp.float32)

        @pl.when(g == 0)
        def _():
            s2_ref[k] = srow

        @pl.when(g > 0)
        def _():
            s2_ref[k] = s2_ref[k] + srow


def _run_blk(k, aggs, inn, params, w1, b1r, w2r):
    zis = _ZIS[k]
    spec, nw = _contrib_spec(k)
    ws = jnp.stack([params['W_%s_%s' % (blk, rl)]
                    for ent in spec for (_, _, _, _, blk, rl) in ent])
    bsum = jnp.stack([
        sum(params['b_%s_%s' % (_ZBLOCK[zi], rl)] for (_, rl) in _CONTRIB[zi])
        for zi in zis]).reshape(len(zis), 1, _D)
    at = jnp.stack([jnp.full((_D,), params['a_' + _ZBLOCK[zi]])
                    for zi in zis]).reshape(len(zis), 1, _D)
    nz = len(zis)
    body = functools.partial(_blk_body, zis, spec, len(aggs))
    return pl.pallas_call(
        body,
        grid=(_N // _TB,),
        in_specs=[
            pl.BlockSpec((a.shape[0], _TB, _D), lambda g: (0, g, 0))
            for a in aggs] + [
            pl.BlockSpec((_TB, _R), lambda g: (g, 0)),
            pl.BlockSpec((nw, _D, _D), lambda g: (0, 0, 0)),
            pl.BlockSpec((nz, 1, _D), lambda g: (0, 0, 0)),
            pl.BlockSpec((nz, 1, _D), lambda g: (0, 0, 0)),
            pl.BlockSpec((_D, _D), lambda g: (0, 0)),
            pl.BlockSpec((1, _D), lambda g: (0, 0)),
            pl.BlockSpec((1, _D), lambda g: (0, 0)),
        ],
        out_specs=[
            pl.BlockSpec((nz, _TB, _D), lambda g: (0, g, 0)),
            pl.BlockSpec((nz, _D), lambda g: (0, 0)),
        ],
        out_shape=[
            jax.ShapeDtypeStruct((nz, _N, _D), jnp.bfloat16),
            jax.ShapeDtypeStruct((nz, _D), jnp.float32),
        ],
    )(*aggs, inn, ws, bsum, at, w1, b1r, w2r)


_PAIRS = [((_ZSTAGE[2 * nt], _ZIS[_ZSTAGE[2 * nt]].index(2 * nt)),
           (_ZSTAGE[2 * nt + 1], _ZIS[_ZSTAGE[2 * nt + 1]].index(2 * nt + 1)))
          for nt in range(5)]


def _comb_body(*refs):
    nst = len(_PART)
    zs = refs[:nst]
    ss = refs[nst:2 * nst]
    outs = refs[2 * nst:]
    for nt, ((a0, i0), (a1, i1)) in enumerate(_PAIRS):
        s0 = ss[a0][i0]
        s1 = ss[a1][i1]
        m = jnp.maximum(s0, s1)
        e0 = jnp.exp((s0 - m) * (1.0 / _N))
        e1 = jnp.exp((s1 - m) * (1.0 / _N))
        b0 = e0 / (e0 + e1)
        b1 = e1 / (e0 + e1)
        outs[nt][...] = (zs[a0][i0].astype(jnp.float32) * b0[None, :]
                         + zs[a1][i1].astype(jnp.float32) * b1[None, :])


def _run_comb(zlist, slist):
    return pl.pallas_call(
        _comb_body,
        grid=(_N // _TB,),
        in_specs=[
            pl.BlockSpec((z.shape[0], _TB, _D), lambda g: (0, g, 0))
            for z in zlist] + [
            pl.BlockSpec((sv.shape[0], _D), lambda g: (0, 0))
            for sv in slist],
        out_specs=[pl.BlockSpec((_TB, _D), lambda g: (g, 0))] * 5,
        out_shape=[jax.ShapeDtypeStruct((_N, _D), jnp.float32)] * 5,
    )(*zlist, *slist)


def kernel(x_drug, x_disease, x_protein, x_gene, x_pathway, params,
           edge_drug_drug, edge_drug_disease, edge_disease_disease,
           edge_drug_protein, edge_protein_protein, edge_protein_gene,
           edge_gene_gene, edge_gene_pathway, edge_pathway_pathway,
           edge_pathway_disease):
    edges = [edge_drug_drug, edge_drug_disease, edge_disease_disease,
             edge_drug_protein, edge_protein_protein, edge_protein_gene,
             edge_gene_gene, edge_gene_pathway, edge_pathway_pathway,
             edge_pathway_disease]
    xlist = [x_drug, x_disease, x_protein, x_gene, x_pathway]
    eraw = jnp.concatenate([e.reshape(-1) for e in edges])

    degp = _run_deg(eraw)
    deg4 = degp.reshape(_R, 2, 16, _N)

    xs_p, inn_p = [], []
    for part in _PART:
        xsk, innk = _run_norm(xlist, deg4, part[0], len(part))
        xs_p.append(xsk.reshape(len(part) * _N, _D))
        inn_p.append(innk[:, 0, :])
    innt = jnp.concatenate(inn_p, axis=0).T

    aggs = [_run_agg(xs_p[pi], eraw, part)
            for pi, part in enumerate(_PART)]

    b1r = params['att_b1'].reshape(1, _D)
    w2r = params['att_w2'].reshape(1, _D)
    w1 = params['att_W1']

    zlist, slist = [], []
    for k in range(len(_PART)):
        zk, sk = _run_blk(k, [aggs[p] for p in _STAGE_PARTS[k]],
                          innt, params, w1, b1r, w2r)
        zlist.append(zk)
        slist.append(sk)
    outs = _run_comb(zlist, slist)
    return tuple(outs)


# speedup vs baseline: 1.0573x; 1.0119x over previous
import functools

import jax
import jax.numpy as jnp
from jax import lax
from jax.experimental import pallas as pl
from jax.experimental.pallas import tpu as pltpu
from jax.experimental.pallas import tpu_sc as plsc

_N = 10000
_D = 128
_E = 160000
_R = 10
_CH = 128
_NCH = _E // _CH
_SIDE = _R * _E

_RELS = ['drug_drug', 'drug_disease', 'disease_disease', 'drug_protein',
         'protein_protein', 'protein_gene', 'gene_gene', 'gene_pathway',
         'pathway_pathway', 'pathway_disease']
_SRC_T = [0, 0, 1, 0, 2, 2, 3, 3, 4, 4]

_ZBLOCK = ['dd', 'dp', 'dd', 'pd', 'dp', 'pg', 'pg', 'gp', 'gp', 'pd']
_CONTRIB = [
    [(0, 'drug_drug')],
    [(0, 'drug_drug')],
    [(1, 'drug_disease'), (2, 'disease_disease')],
    [(9, 'pathway_disease'), (2, 'disease_disease')],
    [(3, 'drug_protein'), (4, 'protein_protein')],
    [(4, 'protein_protein')],
    [(5, 'protein_gene'), (6, 'gene_gene')],
    [(6, 'gene_gene')],
    [(7, 'gene_pathway'), (8, 'pathway_pathway')],
    [(8, 'pathway_pathway')],
]
_WLIST = []
_CONTRIB_IDX = []
for _zi, _lst in enumerate(_CONTRIB):
    _idxs = []
    for (_r, _rl) in _lst:
        _idxs.append((_r, len(_WLIST)))
        _WLIST.append((_ZBLOCK[_zi], _rl))
    _CONTRIB_IDX.append(_idxs)


_CE = 2000


def _deg_body(eraw, degp, bins, idxb0, idxb1, sem_i0, sem_i1):
    idxb = [idxb0, idxb1]
    c = lax.axis_index("c")
    s = lax.axis_index("s")
    zero16 = jnp.zeros((16,), jnp.float32)
    ones16 = jnp.ones((16,), jnp.float32)
    sems = [sem_i0, sem_i1]

    @pl.loop(0, _R * _N // 16)
    def _zero(j):
        bins[pl.ds(j * 16, 16)] = zero16

    nch = _R * (_E // _CE) // 16

    def _issue(t, b):
        r = t // 5
        ch = s + 16 * (t % 5)
        off = (2 * r + c) * _E + ch * _CE
        pltpu.async_copy(eraw.at[pl.ds(off, _CE)], idxb[b], sems[b])

    _issue(0, 0)
    _issue(1, 1)

    @pl.loop(0, nch // 2)
    def _chunks(g):
        for b in range(2):
            t = g * 2 + b
            rn = (t // 5) * _N
            pltpu.make_async_copy(eraw.at[pl.ds(0, _CE)], idxb[b],
                                  sems[b]).wait()

            @pl.loop(0, _CE // 16, unroll=4)
            def _scat(j):
                v = idxb[b][pl.ds(j * 16, 16)] + rn
                plsc.addupdate_scatter(bins, [v], ones16)

            @pl.when(t + 2 < nch)
            def _():
                _issue(t + 2, b)

    w = c * 16 + s
    for r in range(_R):
        pltpu.sync_copy(bins.at[pl.ds(r * _N, _N)],
                        degp.at[pl.ds((r * 32 + w) * _N, _N)])


def _run_deg(ecat):
    mesh = plsc.VectorSubcoreMesh(core_axis_name="c", subcore_axis_name="s",
                                  num_cores=2, num_subcores=16)
    return pl.kernel(
        _deg_body,
        out_type=jax.ShapeDtypeStruct((32 * _R * _N,), jnp.float32),
        mesh=mesh,
        compiler_params=pltpu.CompilerParams(needs_layout_passes=False),
        scratch_types=[
            pltpu.VMEM((_R * _N,), jnp.float32),
            pltpu.VMEM((_CE,), jnp.int32),
            pltpu.VMEM((_CE,), jnp.int32),
            pltpu.SemaphoreType.DMA,
            pltpu.SemaphoreType.DMA,
        ],
    )(ecat)


def _norm_body(r0, types, *refs):
    xrefs = refs[:len(types)]
    deg_ref, xs_ref, inn_ref = refs[len(types):]
    r = pl.program_id(0) + r0
    deg = deg_ref[...]
    outd = jnp.sum(deg[0, 0, :, :], axis=0)
    ind = jnp.sum(deg[0, 1, :, :], axis=0)
    onorm = lax.rsqrt(jnp.maximum(outd, 1.0))
    inorm = lax.rsqrt(jnp.maximum(ind, 1.0))
    st = _src_t_of(r)
    for k, tid in enumerate(types):
        @pl.when(st == tid)
        def _():
            xs_ref[0] = xrefs[k][...] * onorm[:, None]
    inn_ref[0, 0] = inorm


def _src_t_of(r):
    return jnp.where(r >= 4, r // 2, jnp.where(r == 2, 1, 0))


def _run_norm(xlist, deg4, r0, nr):
    types = sorted({_SRC_T[r] for r in range(r0, r0 + nr)})
    return pl.pallas_call(
        functools.partial(_norm_body, r0, types),
        grid=(nr,),
        in_specs=[pl.BlockSpec((_N, _D), lambda r: (0, 0))] * len(types) + [
            pl.BlockSpec((1, 2, 16, _N), lambda r: (r + r0, 0, 0, 0)),
        ],
        out_specs=[
            pl.BlockSpec((1, _N, _D), lambda r: (r, 0, 0)),
            pl.BlockSpec((1, 1, _N), lambda r: (r, 0, 0)),
        ],
        out_shape=[
            jax.ShapeDtypeStruct((nr, _N, _D), jnp.float32),
            jax.ShapeDtypeStruct((nr, 1, _N), jnp.float32),
        ],
    )(*[xlist[t] for t in types], deg4)


def _agg_body(rels, xsf, eraw, agg, acc, idx2, rows, zrow,
              sem_i0, sem_i1, sem_i2, sem_i3, sem_g0, sem_g1,
              sem_s0, sem_s1):
    c = lax.axis_index("c")
    s = lax.axis_index("s")
    zero16 = jnp.zeros((16,), jnp.float32)
    sems_i = [sem_i0, sem_i1, sem_i2, sem_i3]
    sems_g = [sem_g0, sem_g1]
    sems_s = [sem_s0, sem_s1]

    @pl.loop(0, 80)
    def _zz(j):
        for t in range(8):
            zrow[j, pl.ds(t * 16, 16)] = zero16

    def _zero_stripe():
        @pl.loop(0, 8)
        def _zs(t):
            g = s + 16 * t

            @pl.when(g < 125)
            def _():
                pltpu.sync_copy(zrow, acc.at[pl.ds(g * 80, 80)])

    _zero_stripe()
    for i in range(len(rels) // 2):
        p0, p1 = rels[2 * i], rels[2 * i + 1]
        r = p0 + c * (p1 - p0)
        li = 2 * i + c
        plsc.subcore_barrier()

        rn = (r - rels[0]) * _N

        def _issue_idx(t, bi):
            @pl.when(s + 16 * t < _NCH)
            def _():
                off = 2 * r * _E + (s + 16 * t) * _CH
                pltpu.async_copy(eraw.at[pl.ds(off, _CH)], idx2.at[bi, 0],
                                 sems_i[bi])
                pltpu.async_copy(eraw.at[pl.ds(off + _E, _CH)],
                                 idx2.at[bi, 1], sems_i[bi])

        def _wait_adjust_idx(t, bi):
            @pl.when(s + 16 * t < _NCH)
            def _():
                pltpu.make_async_copy(eraw.at[pl.ds(0, _CH)],
                                      idx2.at[bi, 0], sems_i[bi]).wait()
                pltpu.make_async_copy(eraw.at[pl.ds(0, _CH)],
                                      idx2.at[bi, 1], sems_i[bi]).wait()
                for j in range(8):
                    sl = pl.ds(j * 16, 16)
                    idx2[bi, 0, sl] = idx2[bi, 0, sl] + rn

        def _issue_gather(t, bi, bg):
            @pl.when(s + 16 * t < _NCH)
            def _():
                pltpu.async_copy(xsf.at[idx2.at[bi, 0]], rows.at[bg],
                                 sems_g[bg])

        def _wait_gather(t, bg):
            @pl.when(s + 16 * t < _NCH)
            def _():
                pltpu.make_async_copy(xsf.at[pl.ds(0, _CH)], rows.at[bg],
                                      sems_g[bg]).wait()

        def _issue_scatter(t, bi, bg):
            @pl.when(s + 16 * t < _NCH)
            def _():
                pltpu.async_copy(rows.at[bg], acc.at[idx2.at[bi, 1]],
                                 sems_s[bg], add=True)

        def _wait_scatter(t, bg, extra_cond):
            @pl.when(jnp.logical_and(extra_cond, s + 16 * t < _NCH))
            def _():
                pltpu.make_async_copy(xsf.at[pl.ds(0, _CH)], rows.at[bg],
                                      sems_s[bg]).wait()

        _issue_idx(0, 0)
        _issue_idx(1, 1)
        _issue_idx(2, 2)
        _wait_adjust_idx(0, 0)
        _issue_gather(0, 0, 0)

        @pl.loop(0, 21)
        def _chunks(g):
            for u in range(4):
                t = g * 4 + u
                _wait_adjust_idx(t + 1, (u + 1) % 4)
                _wait_gather(t, u % 2)
                _wait_scatter(t - 1, (u + 1) % 2, t >= 1)
                _issue_gather(t + 1, (u + 1) % 4, (u + 1) % 2)
                _issue_scatter(t, u % 4, u % 2)
                _issue_idx(t + 3, (u + 3) % 4)

        plsc.subcore_barrier()

        @pl.loop(0, 8)
        def _drain(t):
            g = s + 16 * t

            @pl.when(g < 125)
            def _():
                sl = pl.ds(g * 80, 80)
                pltpu.sync_copy(acc.at[sl], agg.at[li, sl])

        if i < len(rels) // 2 - 1:
            _zero_stripe()


def _run_agg(xsf, eraw, rels):
    mesh = plsc.VectorSubcoreMesh(core_axis_name="c", subcore_axis_name="s",
                                  num_cores=2, num_subcores=16)
    return pl.kernel(
        functools.partial(_agg_body, rels),
        out_type=jax.ShapeDtypeStruct((len(rels), _N, _D), jnp.float32),
        mesh=mesh,
        compiler_params=pltpu.CompilerParams(needs_layout_passes=False),
        scratch_types=[
            pltpu.VMEM_SHARED((_N, _D), jnp.float32),
            pltpu.VMEM((4, 2, _CH), jnp.int32),
            pltpu.VMEM((2, _CH, _D), jnp.float32),
            pltpu.VMEM((80, _D), jnp.float32),
        ] + [pltpu.SemaphoreType.DMA] * 8,
    )(xsf, eraw)


_TB = 1000

_PART = [[0, 1], [2, 3, 4, 5], [6, 7, 8, 9]]


def _part_of(r):
    for pi, p in enumerate(_PART):
        if r in p:
            return pi
    raise AssertionError


_ZSTAGE = [max(_part_of(r) for (r, _) in _CONTRIB[zi]) for zi in range(10)]
_ZIS = [[zi for zi in range(10) if _ZSTAGE[zi] == k]
        for k in range(len(_PART))]
_STAGE_PARTS = [sorted({_part_of(r) for zi in _ZIS[k]
                        for (r, _) in _CONTRIB[zi]})
                for k in range(len(_PART))]


def _contrib_spec(k):
    parts = _STAGE_PARTS[k]
    spec, nw = [], 0
    for zi in _ZIS[k]:
        ent = []
        for (r, rl) in _CONTRIB[zi]:
            p = _part_of(r)
            ent.append((parts.index(p), _PART[p].index(r), r, nw,
                        _ZBLOCK[zi], rl))
            nw += 1
        spec.append(ent)
    return spec, nw


def _blk_body(zis, spec, n_agg, *refs):
    agg_refs = refs[:n_agg]
    inn_ref, ws_ref, bsum_ref, at_ref, w1_ref, b1_ref, w2_ref = \
        refs[n_agg:n_agg + 7]
    z_ref, s2_ref = refs[n_agg + 7:]
    g = pl.program_id(0)
    w1 = w1_ref[...]
    b1 = b1_ref[...]
    w2r = w2_ref[...]
    sc = {}
    for k in range(len(zis)):
        tot = None
        for (a, lr, gr, wi, _, _) in spec[k]:
            if (a, lr) not in sc:
                sc[(a, lr)] = agg_refs[a][lr] * inn_ref[:, gr][:, None]
            y = jnp.dot(sc[(a, lr)], ws_ref[wi],
                        preferred_element_type=jnp.float32)
            tot = y if tot is None else tot + y
        tot = tot + bsum_ref[k]
        z = jnp.where(tot >= 0, tot, tot * at_ref[k])
        z_ref[k] = z.astype(jnp.bfloat16)
        t = jnp.tanh(jnp.dot(z, w1, preferred_element_type=jnp.float32) + b1)
        sco = jnp.sum(t * w2r)
        srow = jnp.full((_D,), sco, jnp.float32)

        @pl.when(g == 0)
        def _():
            s2_ref[k] = srow

        @pl.when(g > 0)
        def _():
            s2_ref[k] = s2_ref[k] + srow


def _run_blk(k, aggs, inn, params, w1, b1r, w2r):
    zis = _ZIS[k]
    spec, nw = _contrib_spec(k)
    ws = jnp.stack([params['W_%s_%s' % (blk, rl)]
                    for ent in spec for (_, _, _, _, blk, rl) in ent])
    bsum = jnp.stack([
        sum(params['b_%s_%s' % (_ZBLOCK[zi], rl)] for (_, rl) in _CONTRIB[zi])
        for zi in zis]).reshape(len(zis), 1, _D)
    at = jnp.stack([jnp.full((_D,), params['a_' + _ZBLOCK[zi]])
                    for zi in zis]).reshape(len(zis), 1, _D)
    nz = len(zis)
    body = functools.partial(_blk_body, zis, spec, len(aggs))
    return pl.pallas_call(
        body,
        grid=(_N // _TB,),
        in_specs=[
            pl.BlockSpec((a.shape[0], _TB, _D), lambda g: (0, g, 0))
            for a in aggs] + [
            pl.BlockSpec((_TB, _R), lambda g: (g, 0)),
            pl.BlockSpec((nw, _D, _D), lambda g: (0, 0, 0)),
            pl.BlockSpec((nz, 1, _D), lambda g: (0, 0, 0)),
            pl.BlockSpec((nz, 1, _D), lambda g: (0, 0, 0)),
            pl.BlockSpec((_D, _D), lambda g: (0, 0)),
            pl.BlockSpec((1, _D), lambda g: (0, 0)),
            pl.BlockSpec((1, _D), lambda g: (0, 0)),
        ],
        out_specs=[
            pl.BlockSpec((nz, _TB, _D), lambda g: (0, g, 0)),
            pl.BlockSpec((nz, _D), lambda g: (0, 0)),
        ],
        out_shape=[
            jax.ShapeDtypeStruct((nz, _N, _D), jnp.bfloat16),
            jax.ShapeDtypeStruct((nz, _D), jnp.float32),
        ],
    )(*aggs, inn, ws, bsum, at, w1, b1r, w2r)


_PAIRS = [((_ZSTAGE[2 * nt], _ZIS[_ZSTAGE[2 * nt]].index(2 * nt)),
           (_ZSTAGE[2 * nt + 1], _ZIS[_ZSTAGE[2 * nt + 1]].index(2 * nt + 1)))
          for nt in range(5)]


def _comb_body(*refs):
    nst = len(_PART)
    zs = refs[:nst]
    ss = refs[nst:2 * nst]
    outs = refs[2 * nst:]
    for nt, ((a0, i0), (a1, i1)) in enumerate(_PAIRS):
        s0 = ss[a0][i0]
        s1 = ss[a1][i1]
        m = jnp.maximum(s0, s1)
        e0 = jnp.exp((s0 - m) * (1.0 / _N))
        e1 = jnp.exp((s1 - m) * (1.0 / _N))
        b0 = e0 / (e0 + e1)
        b1 = e1 / (e0 + e1)
        outs[nt][...] = (zs[a0][i0].astype(jnp.float32) * b0[None, :]
                         + zs[a1][i1].astype(jnp.float32) * b1[None, :])


def _run_comb(zlist, slist):
    return pl.pallas_call(
        _comb_body,
        grid=(_N // _TB,),
        in_specs=[
            pl.BlockSpec((z.shape[0], _TB, _D), lambda g: (0, g, 0))
            for z in zlist] + [
            pl.BlockSpec((sv.shape[0], _D), lambda g: (0, 0))
            for sv in slist],
        out_specs=[pl.BlockSpec((_TB, _D), lambda g: (g, 0))] * 5,
        out_shape=[jax.ShapeDtypeStruct((_N, _D), jnp.float32)] * 5,
    )(*zlist, *slist)


def kernel(x_drug, x_disease, x_protein, x_gene, x_pathway, params,
           edge_drug_drug, edge_drug_disease, edge_disease_disease,
           edge_drug_protein, edge_protein_protein, edge_protein_gene,
           edge_gene_gene, edge_gene_pathway, edge_pathway_pathway,
           edge_pathway_disease):
    edges = [edge_drug_drug, edge_drug_disease, edge_disease_disease,
             edge_drug_protein, edge_protein_protein, edge_protein_gene,
             edge_gene_gene, edge_gene_pathway, edge_pathway_pathway,
             edge_pathway_disease]
    xlist = [x_drug, x_disease, x_protein, x_gene, x_pathway]
    eraw = jnp.concatenate([e.reshape(-1) for e in edges])

    degp = _run_deg(eraw)
    deg4 = degp.reshape(_R, 2, 16, _N)

    xs_p, inn_p = [], []
    for part in _PART:
        xsk, innk = _run_norm(xlist, deg4, part[0], len(part))
        xs_p.append(xsk.reshape(len(part) * _N, _D))
        inn_p.append(innk[:, 0, :])
    innt = jnp.concatenate(inn_p, axis=0).T

    aggs = [_run_agg(xs_p[pi], eraw, part)
            for pi, part in enumerate(_PART)]

    b1r = params['att_b1'].reshape(1, _D)
    w2r = params['att_w2'].reshape(1, _D)
    w1 = params['att_W1']

    zlist, slist = [], []
    for k in range(len(_PART)):
        zk, sk = _run_blk(k, [aggs[p] for p in _STAGE_PARTS[k]],
                          innt, params, w1, b1r, w2r)
        zlist.append(zk)
        slist.append(sk)
    outs = _run_comb(zlist, slist)
    return tuple(outs)


# speedup vs baseline: 1.0771x; 1.0187x over previous
import functools

import jax
import jax.numpy as jnp
from jax import lax
from jax.experimental import pallas as pl
from jax.experimental.pallas import tpu as pltpu
from jax.experimental.pallas import tpu_sc as plsc

_N = 10000
_D = 128
_E = 160000
_R = 10
_CH = 128
_NCH = _E // _CH
_SIDE = _R * _E

_RELS = ['drug_drug', 'drug_disease', 'disease_disease', 'drug_protein',
         'protein_protein', 'protein_gene', 'gene_gene', 'gene_pathway',
         'pathway_pathway', 'pathway_disease']
_SRC_T = [0, 0, 1, 0, 2, 2, 3, 3, 4, 4]

_ZBLOCK = ['dd', 'dp', 'dd', 'pd', 'dp', 'pg', 'pg', 'gp', 'gp', 'pd']
_CONTRIB = [
    [(0, 'drug_drug')],
    [(0, 'drug_drug')],
    [(1, 'drug_disease'), (2, 'disease_disease')],
    [(9, 'pathway_disease'), (2, 'disease_disease')],
    [(3, 'drug_protein'), (4, 'protein_protein')],
    [(4, 'protein_protein')],
    [(5, 'protein_gene'), (6, 'gene_gene')],
    [(6, 'gene_gene')],
    [(7, 'gene_pathway'), (8, 'pathway_pathway')],
    [(8, 'pathway_pathway')],
]
_WLIST = []
_CONTRIB_IDX = []
for _zi, _lst in enumerate(_CONTRIB):
    _idxs = []
    for (_r, _rl) in _lst:
        _idxs.append((_r, len(_WLIST)))
        _WLIST.append((_ZBLOCK[_zi], _rl))
    _CONTRIB_IDX.append(_idxs)


_CE = 2000


def _deg_body(eraw, degp, bins, idxb0, idxb1, sem_i0, sem_i1):
    idxb = [idxb0, idxb1]
    c = lax.axis_index("c")
    s = lax.axis_index("s")
    zero16 = jnp.zeros((16,), jnp.float32)
    ones16 = jnp.ones((16,), jnp.float32)
    sems = [sem_i0, sem_i1]

    @pl.loop(0, _R * _N // 16)
    def _zero(j):
        bins[pl.ds(j * 16, 16)] = zero16

    nch = _R * (_E // _CE) // 16

    def _issue(t, b):
        r = t // 5
        ch = s + 16 * (t % 5)
        off = (2 * r + c) * _E + ch * _CE
        pltpu.async_copy(eraw.at[pl.ds(off, _CE)], idxb[b], sems[b])

    _issue(0, 0)
    _issue(1, 1)

    @pl.loop(0, nch // 2)
    def _chunks(g):
        for b in range(2):
            t = g * 2 + b
            rn = (t // 5) * _N
            pltpu.make_async_copy(eraw.at[pl.ds(0, _CE)], idxb[b],
                                  sems[b]).wait()

            @pl.loop(0, _CE // 16, unroll=8)
            def _scat(j):
                v = idxb[b][pl.ds(j * 16, 16)] + rn
                plsc.addupdate_scatter(bins, [v], ones16)

            @pl.when(t + 2 < nch)
            def _():
                _issue(t + 2, b)

    w = c * 16 + s
    for r in range(_R):
        pltpu.sync_copy(bins.at[pl.ds(r * _N, _N)],
                        degp.at[pl.ds((r * 32 + w) * _N, _N)])


def _run_deg(ecat):
    mesh = plsc.VectorSubcoreMesh(core_axis_name="c", subcore_axis_name="s",
                                  num_cores=2, num_subcores=16)
    return pl.kernel(
        _deg_body,
        out_type=jax.ShapeDtypeStruct((32 * _R * _N,), jnp.float32),
        mesh=mesh,
        compiler_params=pltpu.CompilerParams(needs_layout_passes=False),
        scratch_types=[
            pltpu.VMEM((_R * _N,), jnp.float32),
            pltpu.VMEM((_CE,), jnp.int32),
            pltpu.VMEM((_CE,), jnp.int32),
            pltpu.SemaphoreType.DMA,
            pltpu.SemaphoreType.DMA,
        ],
    )(ecat)


def _norm_body(r0, types, *refs):
    xrefs = refs[:len(types)]
    deg_ref, xs_ref, inn_ref = refs[len(types):]
    r = pl.program_id(0) + r0
    deg = deg_ref[...]
    outd = jnp.sum(deg[0, 0, :, :], axis=0)
    ind = jnp.sum(deg[0, 1, :, :], axis=0)
    onorm = lax.rsqrt(jnp.maximum(outd, 1.0))
    inorm = lax.rsqrt(jnp.maximum(ind, 1.0))
    st = _src_t_of(r)
    for k, tid in enumerate(types):
        @pl.when(st == tid)
        def _():
            xs_ref[0] = xrefs[k][...] * onorm[:, None]
    inn_ref[0, 0] = inorm


def _src_t_of(r):
    return jnp.where(r >= 4, r // 2, jnp.where(r == 2, 1, 0))


def _run_norm(xlist, deg4, r0, nr):
    types = sorted({_SRC_T[r] for r in range(r0, r0 + nr)})
    return pl.pallas_call(
        functools.partial(_norm_body, r0, types),
        grid=(nr,),
        in_specs=[pl.BlockSpec((_N, _D), lambda r: (0, 0))] * len(types) + [
            pl.BlockSpec((1, 2, 16, _N), lambda r: (r + r0, 0, 0, 0)),
        ],
        out_specs=[
            pl.BlockSpec((1, _N, _D), lambda r: (r, 0, 0)),
            pl.BlockSpec((1, 1, _N), lambda r: (r, 0, 0)),
        ],
        out_shape=[
            jax.ShapeDtypeStruct((nr, _N, _D), jnp.float32),
            jax.ShapeDtypeStruct((nr, 1, _N), jnp.float32),
        ],
    )(*[xlist[t] for t in types], deg4)


def _agg_body(rels, xsf, eraw, agg, acc, idx2, rows, zrow,
              sem_i0, sem_i1, sem_i2, sem_i3, sem_g0, sem_g1,
              sem_s0, sem_s1):
    c = lax.axis_index("c")
    s = lax.axis_index("s")
    zero16 = jnp.zeros((16,), jnp.float32)
    sems_i = [sem_i0, sem_i1, sem_i2, sem_i3]
    sems_g = [sem_g0, sem_g1]
    sems_s = [sem_s0, sem_s1]

    @pl.loop(0, 80)
    def _zz(j):
        for t in range(8):
            zrow[j, pl.ds(t * 16, 16)] = zero16

    def _zero_stripe():
        @pl.loop(0, 8)
        def _zs(t):
            g = s + 16 * t

            @pl.when(g < 125)
            def _():
                pltpu.sync_copy(zrow, acc.at[pl.ds(g * 80, 80)])

    _zero_stripe()
    for i in range(len(rels) // 2):
        p0, p1 = rels[2 * i], rels[2 * i + 1]
        r = p0 + c * (p1 - p0)
        li = 2 * i + c
        plsc.subcore_barrier()

        rn = (r - rels[0]) * _N

        def _issue_idx(t, bi):
            @pl.when(s + 16 * t < _NCH)
            def _():
                off = 2 * r * _E + (s + 16 * t) * _CH
                pltpu.async_copy(eraw.at[pl.ds(off, _CH)], idx2.at[bi, 0],
                                 sems_i[bi])
                pltpu.async_copy(eraw.at[pl.ds(off + _E, _CH)],
                                 idx2.at[bi, 1], sems_i[bi])

        def _wait_adjust_idx(t, bi):
            @pl.when(s + 16 * t < _NCH)
            def _():
                pltpu.make_async_copy(eraw.at[pl.ds(0, _CH)],
                                      idx2.at[bi, 0], sems_i[bi]).wait()
                pltpu.make_async_copy(eraw.at[pl.ds(0, _CH)],
                                      idx2.at[bi, 1], sems_i[bi]).wait()
                for j in range(8):
                    sl = pl.ds(j * 16, 16)
                    idx2[bi, 0, sl] = idx2[bi, 0, sl] + rn

        def _issue_gather(t, bi, bg):
            @pl.when(s + 16 * t < _NCH)
            def _():
                pltpu.async_copy(xsf.at[idx2.at[bi, 0]], rows.at[bg],
                                 sems_g[bg])

        def _wait_gather(t, bg):
            @pl.when(s + 16 * t < _NCH)
            def _():
                pltpu.make_async_copy(xsf.at[pl.ds(0, _CH)], rows.at[bg],
                                      sems_g[bg]).wait()

        def _issue_scatter(t, bi, bg):
            @pl.when(s + 16 * t < _NCH)
            def _():
                pltpu.async_copy(rows.at[bg], acc.at[idx2.at[bi, 1]],
                                 sems_s[bg], add=True)

        def _wait_scatter(t, bg, extra_cond):
            @pl.when(jnp.logical_and(extra_cond, s + 16 * t < _NCH))
            def _():
                pltpu.make_async_copy(xsf.at[pl.ds(0, _CH)], rows.at[bg],
                                      sems_s[bg]).wait()

        _issue_idx(0, 0)
        _issue_idx(1, 1)
        _issue_idx(2, 2)
        _wait_adjust_idx(0, 0)
        _issue_gather(0, 0, 0)

        @pl.loop(0, 21)
        def _chunks(g):
            for u in range(4):
                t = g * 4 + u
                _wait_adjust_idx(t + 1, (u + 1) % 4)
                _wait_gather(t, u % 2)
                _wait_scatter(t - 1, (u + 1) % 2, t >= 1)
                _issue_gather(t + 1, (u + 1) % 4, (u + 1) % 2)
                _issue_scatter(t, u % 4, u % 2)
                _issue_idx(t + 3, (u + 3) % 4)

        plsc.subcore_barrier()

        @pl.loop(0, 8)
        def _drain(t):
            g = s + 16 * t

            @pl.when(g < 125)
            def _():
                sl = pl.ds(g * 80, 80)
                pltpu.sync_copy(acc.at[sl], agg.at[li, sl])

        if i < len(rels) // 2 - 1:
            _zero_stripe()


def _run_agg(xsf, eraw, rels):
    mesh = plsc.VectorSubcoreMesh(core_axis_name="c", subcore_axis_name="s",
                                  num_cores=2, num_subcores=16)
    return pl.kernel(
        functools.partial(_agg_body, rels),
        out_type=jax.ShapeDtypeStruct((len(rels), _N, _D), jnp.float32),
        mesh=mesh,
        compiler_params=pltpu.CompilerParams(needs_layout_passes=False),
        scratch_types=[
            pltpu.VMEM_SHARED((_N, _D), jnp.float32),
            pltpu.VMEM((4, 2, _CH), jnp.int32),
            pltpu.VMEM((2, _CH, _D), jnp.float32),
            pltpu.VMEM((80, _D), jnp.float32),
        ] + [pltpu.SemaphoreType.DMA] * 8,
    )(xsf, eraw)


_TB = 2000

_PART = [[0, 1], [2, 3, 4, 5], [6, 7, 8, 9]]


def _part_of(r):
    for pi, p in enumerate(_PART):
        if r in p:
            return pi
    raise AssertionError


_ZSTAGE = [max(_part_of(r) for (r, _) in _CONTRIB[zi]) for zi in range(10)]
_ZIS = [[zi for zi in range(10) if _ZSTAGE[zi] == k]
        for k in range(len(_PART))]
_STAGE_PARTS = [sorted({_part_of(r) for zi in _ZIS[k]
                        for (r, _) in _CONTRIB[zi]})
                for k in range(len(_PART))]


def _contrib_spec(k):
    parts = _STAGE_PARTS[k]
    spec, nw = [], 0
    for zi in _ZIS[k]:
        ent = []
        for (r, rl) in _CONTRIB[zi]:
            p = _part_of(r)
            ent.append((parts.index(p), _PART[p].index(r), r, nw,
                        _ZBLOCK[zi], rl))
            nw += 1
        spec.append(ent)
    return spec, nw


def _blk_body(zis, spec, n_agg, *refs):
    agg_refs = refs[:n_agg]
    inn_ref, ws_ref, bsum_ref, at_ref, w1_ref, b1_ref, w2_ref = \
        refs[n_agg:n_agg + 7]
    z_ref, s2_ref = refs[n_agg + 7:]
    g = pl.program_id(0)
    w1 = w1_ref[...]
    b1 = b1_ref[...]
    w2r = w2_ref[...]
    sc = {}
    for k in range(len(zis)):
        tot = None
        for (a, lr, gr, wi, _, _) in spec[k]:
            if (a, lr) not in sc:
                sc[(a, lr)] = agg_refs[a][lr] * inn_ref[:, gr][:, None]
            y = jnp.dot(sc[(a, lr)], ws_ref[wi],
                        preferred_element_type=jnp.float32)
            tot = y if tot is None else tot + y
        tot = tot + bsum_ref[k]
        z = jnp.where(tot >= 0, tot, tot * at_ref[k])
        z_ref[k] = z.astype(jnp.bfloat16)
        t = jnp.tanh(jnp.dot(z, w1, preferred_element_type=jnp.float32) + b1)
        sco = jnp.sum(t * w2r)
        srow = jnp.full((_D,), sco, jnp.float32)

        @pl.when(g == 0)
        def _():
            s2_ref[k] = srow

        @pl.when(g > 0)
        def _():
            s2_ref[k] = s2_ref[k] + srow


def _run_blk(k, aggs, inn, params, w1, b1r, w2r):
    zis = _ZIS[k]
    spec, nw = _contrib_spec(k)
    ws = jnp.stack([params['W_%s_%s' % (blk, rl)]
                    for ent in spec for (_, _, _, _, blk, rl) in ent])
    bsum = jnp.stack([
        sum(params['b_%s_%s' % (_ZBLOCK[zi], rl)] for (_, rl) in _CONTRIB[zi])
        for zi in zis]).reshape(len(zis), 1, _D)
    at = jnp.stack([jnp.full((_D,), params['a_' + _ZBLOCK[zi]])
                    for zi in zis]).reshape(len(zis), 1, _D)
    nz = len(zis)
    body = functools.partial(_blk_body, zis, spec, len(aggs))
    return pl.pallas_call(
        body,
        grid=(_N // _TB,),
        in_specs=[
            pl.BlockSpec((a.shape[0], _TB, _D), lambda g: (0, g, 0))
            for a in aggs] + [
            pl.BlockSpec((_TB, _R), lambda g: (g, 0)),
            pl.BlockSpec((nw, _D, _D), lambda g: (0, 0, 0)),
            pl.BlockSpec((nz, 1, _D), lambda g: (0, 0, 0)),
            pl.BlockSpec((nz, 1, _D), lambda g: (0, 0, 0)),
            pl.BlockSpec((_D, _D), lambda g: (0, 0)),
            pl.BlockSpec((1, _D), lambda g: (0, 0)),
            pl.BlockSpec((1, _D), lambda g: (0, 0)),
        ],
        out_specs=[
            pl.BlockSpec((nz, _TB, _D), lambda g: (0, g, 0)),
            pl.BlockSpec((nz, _D), lambda g: (0, 0)),
        ],
        out_shape=[
            jax.ShapeDtypeStruct((nz, _N, _D), jnp.bfloat16),
            jax.ShapeDtypeStruct((nz, _D), jnp.float32),
        ],
    )(*aggs, inn, ws, bsum, at, w1, b1r, w2r)


_PAIRS = [((_ZSTAGE[2 * nt], _ZIS[_ZSTAGE[2 * nt]].index(2 * nt)),
           (_ZSTAGE[2 * nt + 1], _ZIS[_ZSTAGE[2 * nt + 1]].index(2 * nt + 1)))
          for nt in range(5)]


def _comb_body(*refs):
    nst = len(_PART)
    zs = refs[:nst]
    ss = refs[nst:2 * nst]
    outs = refs[2 * nst:]
    for nt, ((a0, i0), (a1, i1)) in enumerate(_PAIRS):
        s0 = ss[a0][i0]
        s1 = ss[a1][i1]
        m = jnp.maximum(s0, s1)
        e0 = jnp.exp((s0 - m) * (1.0 / _N))
        e1 = jnp.exp((s1 - m) * (1.0 / _N))
        b0 = e0 / (e0 + e1)
        b1 = e1 / (e0 + e1)
        outs[nt][...] = (zs[a0][i0].astype(jnp.float32) * b0[None, :]
                         + zs[a1][i1].astype(jnp.float32) * b1[None, :])


def _run_comb(zlist, slist):
    return pl.pallas_call(
        _comb_body,
        grid=(_N // _TB,),
        in_specs=[
            pl.BlockSpec((z.shape[0], _TB, _D), lambda g: (0, g, 0))
            for z in zlist] + [
            pl.BlockSpec((sv.shape[0], _D), lambda g: (0, 0))
            for sv in slist],
        out_specs=[pl.BlockSpec((_TB, _D), lambda g: (g, 0))] * 5,
        out_shape=[jax.ShapeDtypeStruct((_N, _D), jnp.float32)] * 5,
    )(*zlist, *slist)


def kernel(x_drug, x_disease, x_protein, x_gene, x_pathway, params,
           edge_drug_drug, edge_drug_disease, edge_disease_disease,
           edge_drug_protein, edge_protein_protein, edge_protein_gene,
           edge_gene_gene, edge_gene_pathway, edge_pathway_pathway,
           edge_pathway_disease):
    edges = [edge_drug_drug, edge_drug_disease, edge_disease_disease,
             edge_drug_protein, edge_protein_protein, edge_protein_gene,
             edge_gene_gene, edge_gene_pathway, edge_pathway_pathway,
             edge_pathway_disease]
    xlist = [x_drug, x_disease, x_protein, x_gene, x_pathway]
    eraw = jnp.concatenate([e.reshape(-1) for e in edges])

    degp = _run_deg(eraw)
    deg4 = degp.reshape(_R, 2, 16, _N)

    xs_p, inn_p = [], []
    for part in _PART:
        xsk, innk = _run_norm(xlist, deg4, part[0], len(part))
        xs_p.append(xsk.reshape(len(part) * _N, _D))
        inn_p.append(innk[:, 0, :])
    innt = jnp.concatenate(inn_p, axis=0).T

    aggs = [_run_agg(xs_p[pi], eraw, part)
            for pi, part in enumerate(_PART)]

    b1r = params['att_b1'].reshape(1, _D)
    w2r = params['att_w2'].reshape(1, _D)
    w1 = params['att_W1']

    zlist, slist = [], []
    for k in range(len(_PART)):
        zk, sk = _run_blk(k, [aggs[p] for p in _STAGE_PARTS[k]],
                          innt, params, w1, b1r, w2r)
        zlist.append(zk)
        slist.append(sk)
    outs = _run_comb(zlist, slist)
    return tuple(outs)
